# baseline (speedup 1.0000x reference)
import functools

import jax
import jax.numpy as jnp
from jax import lax
from jax.experimental import pallas as pl
from jax.experimental.pallas import tpu as pltpu
from jax.experimental.pallas import tpu_sc as plsc

_N = 10000
_E = 320000
_H = 128
_NC = 2
_NS = 16
_NW = _NC * _NS
_CH = 80

_f32 = jnp.float32


def _dot(a, b):
    return jnp.dot(a, b, preferred_element_type=_f32)


def _mm_bias_kernel(x_ref, w_ref, b_ref, o_ref):
    o_ref[...] = _dot(x_ref[...], w_ref[...]) + b_ref[...]


def _mm(x, w, b, blk):
    m, k = x.shape
    n = w.shape[1]
    return pl.pallas_call(
        _mm_bias_kernel,
        grid=(m // blk,),
        in_specs=[
            pl.BlockSpec((blk, k), lambda i: (i, 0)),
            pl.BlockSpec((k, n), lambda i: (0, 0)),
            pl.BlockSpec((1, n), lambda i: (0, 0)),
        ],
        out_specs=pl.BlockSpec((blk, n), lambda i: (i, 0)),
        out_shape=jax.ShapeDtypeStruct((m, n), _f32),
    )(x, w, b)


def _node_tf_kernel(h_ref, wa, ba, wb, bb, wd, bd, we, be,
                    ah_ref, db_ref, eh_ref):
    h = h_ref[...]
    ah_ref[...] = _dot(h, wa[...]) + ba[...]
    db_ref[:, :_H] = _dot(h, wd[...]) + bd[...]
    db_ref[:, _H:] = _dot(h, wb[...]) + bb[...]
    eh_ref[...] = _dot(h, we[...]) + be[...]


def _node_tf(h, lp, blk=2000):
    wspec = pl.BlockSpec((_H, _H), lambda i: (0, 0))
    bspec = pl.BlockSpec((1, _H), lambda i: (0, 0))
    r2 = lambda b: b.reshape(1, _H)
    return pl.pallas_call(
        _node_tf_kernel,
        grid=(_N // blk,),
        in_specs=[pl.BlockSpec((blk, _H), lambda i: (i, 0))]
        + [wspec, bspec] * 4,
        out_specs=[
            pl.BlockSpec((blk, _H), lambda i: (i, 0)),
            pl.BlockSpec((blk, 2 * _H), lambda i: (i, 0)),
            pl.BlockSpec((blk, _H), lambda i: (i, 0)),
        ],
        out_shape=[
            jax.ShapeDtypeStruct((_N, _H), _f32),
            jax.ShapeDtypeStruct((_N, 2 * _H), _f32),
            jax.ShapeDtypeStruct((_N, _H), _f32),
        ],
    )(h, lp['A'][0], r2(lp['A'][1]), lp['B'][0], r2(lp['B'][1]),
      lp['D'][0], r2(lp['D'][1]), lp['E'][0], r2(lp['E'][1]))


def _edge_fuse_kernel(e_ref, gm_ref, gb_ref, wc, bc,
                      eo_ref, sig_ref, np_ref):
    ce = _dot(e_ref[...], wc[...]) + bc[...]
    en = ce + gm_ref[...]
    sig = jax.nn.sigmoid(en)
    eo_ref[...] = e_ref[...] + jnp.maximum(en, 0.0)
    sig_ref[...] = sig
    np_ref[...] = sig * gb_ref[...]


def _edge_fuse(e, gm, gb, wc, bc, blk=1280):
    espec = pl.BlockSpec((blk, _H), lambda i: (i, 0))
    return pl.pallas_call(
        _edge_fuse_kernel,
        grid=(_E // blk,),
        in_specs=[
            espec, espec, espec,
            pl.BlockSpec((_H, _H), lambda i: (0, 0)),
            pl.BlockSpec((1, _H), lambda i: (0, 0)),
        ],
        out_specs=[espec, espec, espec],
        out_shape=[jax.ShapeDtypeStruct((_E, _H), _f32)] * 3,
    )(e, gm, gb, wc, bc.reshape(1, _H))


def _h_update_kernel(h_ref, ah_ref, num_ref, den_ref, o_ref):
    o_ref[...] = h_ref[...] + jnp.maximum(
        ah_ref[...] + num_ref[...] / (den_ref[...] + 1e-6), 0.0)


def _h_update(h, ah, num, den, blk=2000):
    spec = pl.BlockSpec((blk, _H), lambda i: (i, 0))
    return pl.pallas_call(
        _h_update_kernel,
        grid=(_N // blk,),
        in_specs=[spec] * 4,
        out_specs=spec,
        out_shape=jax.ShapeDtypeStruct((_N, _H), _f32),
    )(h, ah, num, den)


def _node_ro_kernel(h_ref, w1, b1, w2, b2, w3, b3, wea, web, beb,
                    hn_ref, p1_ref, p2_ref):
    h = h_ref[...]
    t = jnp.maximum(_dot(h, w1[...]) + b1[...], 0.0)
    t = jnp.maximum(_dot(t, w2[...]) + b2[...], 0.0)
    hn_ref[...] = _dot(t, w3[...]) + b3[...]
    p1_ref[...] = _dot(h, wea[...])
    p2_ref[...] = _dot(h, web[...]) + beb[...]


def _node_ro(h, mlp_n, wea, web, beb, blk=2000):
    specs = []
    args = [h]
    for (w, b) in mlp_n:
        k, n = w.shape
        specs += [pl.BlockSpec((k, n), lambda i: (0, 0)),
                  pl.BlockSpec((1, n), lambda i: (0, 0))]
        args += [w, b.reshape(1, n)]
    specs += [pl.BlockSpec((_H, _H), lambda i: (0, 0))] * 2
    specs += [pl.BlockSpec((1, _H), lambda i: (0, 0))]
    args += [wea, web, beb.reshape(1, _H)]
    hspec = pl.BlockSpec((blk, _H), lambda i: (i, 0))
    return pl.pallas_call(
        _node_ro_kernel,
        grid=(_N // blk,),
        in_specs=[hspec] + specs,
        out_specs=[hspec, hspec, hspec],
        out_shape=[jax.ShapeDtypeStruct((_N, _H), _f32)] * 3,
    )(*args)


def _edge_mlp_kernel(g_ref, w2, b2, w3, b3, o_ref):
    t = jnp.maximum(_dot(g_ref[...], w2[...]) + b2[...], 0.0)
    o_ref[...] = _dot(t, w3[...]) + b3[...]


def _edge_mlp(g, l2, l3, blk=1280):
    w2, b2 = l2
    w3, b3 = l3
    espec = pl.BlockSpec((blk, _H), lambda i: (i, 0))
    return pl.pallas_call(
        _edge_mlp_kernel,
        grid=(_E // blk,),
        in_specs=[
            espec,
            pl.BlockSpec(w2.shape, lambda i: (0, 0)),
            pl.BlockSpec((1, w2.shape[1]), lambda i: (0, 0)),
            pl.BlockSpec(w3.shape, lambda i: (0, 0)),
            pl.BlockSpec((1, w3.shape[1]), lambda i: (0, 0)),
        ],
        out_specs=espec,
        out_shape=jax.ShapeDtypeStruct((_E, _H), _f32),
    )(g, w2, b2.reshape(1, -1), w3, b3.reshape(1, -1))


_MESH = plsc.VectorSubcoreMesh(core_axis_name="c", subcore_axis_name="s")


def _make_gather2(mode):
    d1 = 2 * _H if mode == 'layer' else _H
    per_w = _E // _NW
    n_chunks = per_w // _CH

    def body(t1, t2, i1_hbm, i2_hbm, *refs):
        if mode == 'layer':
            (o1, o2, i1_v, i2_v, r1_v, r2_v,
             si0, si1, sg0, sg1, so0, so1) = refs
        else:
            (o1, i1_v, i2_v, r1_v, r2_v,
             si0, si1, sg0, sg1, so0, so1) = refs
        wid = lax.axis_index("s") * _NC + lax.axis_index("c")
        base = wid * per_w
        si = (si0, si1)
        sg = (sg0, sg1)
        so = (so0, so1)

        def idx_load(i, b, sem_fn=pltpu.async_copy):
            off = pl.multiple_of(base + i * _CH, 8)
            sem_fn(i1_hbm.at[pl.ds(off, _CH)], i1_v.at[b], si[b])
            sem_fn(i2_hbm.at[pl.ds(off, _CH)], i2_v.at[b], si[b])

        def idx_wait(i, b):
            off = pl.multiple_of(base + i * _CH, 8)
            pltpu.make_async_copy(
                i1_hbm.at[pl.ds(off, _CH)], i1_v.at[b], si[b]).wait()
            pltpu.make_async_copy(
                i2_hbm.at[pl.ds(off, _CH)], i2_v.at[b], si[b]).wait()

        def gather_issue(b):
            pltpu.async_copy(t1.at[i1_v.at[b]], r1_v.at[b], sg[b])
            pltpu.async_copy(t2.at[i2_v.at[b]], r2_v.at[b], sg[b])

        def gather_wait(b):
            pltpu.make_async_copy(t1.at[i1_v.at[b]], r1_v.at[b], sg[b]).wait()
            pltpu.make_async_copy(t2.at[i2_v.at[b]], r2_v.at[b], sg[b]).wait()

        def combine(b):
            @pl.loop(0, _CH, unroll=4)
            def _(r):
                for j in range(_H // 16):
                    sl = pl.ds(j * 16, 16)
                    x = r1_v[b, r, sl] + r2_v[b, r, sl]
                    if mode == 'readout':
                        x = jnp.maximum(x, 0.0)
                    r2_v[b, r, sl] = x

        def write_issue(i, b):
            off = pl.multiple_of(base + i * _CH, 8)
            pltpu.async_copy(r2_v.at[b], o1.at[pl.ds(off, _CH)], so[b])
            if mode == 'layer':
                pltpu.async_copy(r1_v.at[b, slice(None), pl.ds(_H, _H)],
                                 o2.at[pl.ds(off, _CH)], so[b])

        def write_wait(i, b):
            off = pl.multiple_of(base + i * _CH, 8)
            pltpu.make_async_copy(
                r2_v.at[b], o1.at[pl.ds(off, _CH)], so[b]).wait()
            if mode == 'layer':
                pltpu.make_async_copy(
                    r1_v.at[b, slice(None), pl.ds(_H, _H)],
                    o2.at[pl.ds(off, _CH)], so[b]).wait()

        def maybe(cond, fn):
            if cond is True:
                fn()
            elif cond is not False:
                pl.when(cond)(fn)

        def stage(i, b, has_prev, has_prev2, has_next):
            maybe(has_prev2, lambda: write_wait(i - 2, b))
            idx_wait(i, b)
            gather_issue(b)

            def drain_prev():
                gather_wait(1 - b)
                combine(1 - b)
                write_issue(i - 1, 1 - b)
            maybe(has_prev, drain_prev)
            maybe(has_next, lambda: idx_load(i + 1, 1 - b))

        idx_load(0, 0)

        @pl.loop(0, n_chunks // 2)
        def _(j):
            i0 = j * 2
            stage(i0, 0, j > 0, j > 0, True)
            stage(i0 + 1, 1, True, j > 0, i0 + 2 < n_chunks)

        last = n_chunks - 1
        if n_chunks % 2 == 1:
            write_wait(last - 2, 0)
            idx_wait(last, 0)
            gather_issue(0)
            gather_wait(1)
            combine(1)
            write_issue(last - 1, 1)
            gather_wait(0)
            combine(0)
            write_issue(last, 0)
            write_wait(last - 1, 1)
            write_wait(last, 0)
        else:
            gather_wait(1)
            combine(1)
            write_issue(last, 1)
            write_wait(last - 1, 0)
            write_wait(last, 1)

    out_struct = jax.ShapeDtypeStruct((_E, _H), _f32)
    return pl.kernel(
        body,
        out_type=((out_struct, out_struct) if mode == 'layer'
                  else out_struct),
        mesh=_MESH,
        scratch_types=[
            pltpu.VMEM((2, _CH), jnp.int32),
            pltpu.VMEM((2, _CH), jnp.int32),
            pltpu.VMEM((2, _CH, d1), _f32),
            pltpu.VMEM((2, _CH, _H), _f32),
        ] + [pltpu.SemaphoreType.DMA] * 6,
    )


_gather_db_e = _make_gather2('layer')
_gather_p1_p2 = _make_gather2('readout')


_NROWS = 632
_N_PAD = _NROWS * _NS


def _scatter2_body(np_hbm, sig_hbm, dst_hbm, zero_hbm, num_hbm, den_hbm,
                   idx_v, pay_v, acc_sh, si0, si1, sp0, sp1, ss0, ss1):
    cid = lax.axis_index("c")
    sid = lax.axis_index("s")
    nrows = _NROWS
    rows0 = sid * nrows
    per_tile = _E // _NS
    ebase = sid * per_tile

    pltpu.sync_copy(zero_hbm.at[pl.ds(rows0, nrows)],
                    acc_sh.at[pl.ds(rows0, nrows)])
    plsc.subcore_barrier()

    n_chunks = per_tile // _CH

    def scatter_from(src_hbm):
        si = (si0, si1)
        sp = (sp0, sp1)
        ss = (ss0, ss1)

        def load(i, b):
            off = pl.multiple_of(ebase + i * _CH, 8)
            pltpu.async_copy(dst_hbm.at[pl.ds(off, _CH)], idx_v.at[b], si[b])
            pltpu.async_copy(src_hbm.at[pl.ds(off, _CH)], pay_v.at[b], sp[b])

        def load_wait(i, b):
            off = pl.multiple_of(ebase + i * _CH, 8)
            pltpu.make_async_copy(
                dst_hbm.at[pl.ds(off, _CH)], idx_v.at[b], si[b]).wait()
            pltpu.make_async_copy(
                src_hbm.at[pl.ds(off, _CH)], pay_v.at[b], sp[b]).wait()

        def scat_issue(b):
            pltpu.async_copy(pay_v.at[b], acc_sh.at[idx_v.at[b]], ss[b],
                             add=True)

        def scat_wait(b):
            pltpu.make_async_copy(pay_v.at[b], acc_sh.at[idx_v.at[b]],
                                  ss[b]).wait()

        def maybe(cond, fn):
            if cond is True:
                fn()
            elif cond is not False:
                pl.when(cond)(fn)

        def stage(i, b, has_prev, has_next):
            load_wait(i, b)
            scat_issue(b)
            maybe(has_prev, lambda: scat_wait(1 - b))
            maybe(has_next, lambda: load(i + 1, 1 - b))

        load(0, 0)

        @pl.loop(0, n_chunks // 2)
        def _(j):
            i0 = j * 2
            stage(i0, 0, j > 0, True)
            stage(i0 + 1, 1, True, i0 + 2 < n_chunks)

        scat_wait(1)

    @pl.when(cid == 0)
    def _():
        scatter_from(np_hbm)

    @pl.when(cid == 1)
    def _():
        scatter_from(sig_hbm)

    plsc.subcore_barrier()

    @pl.when(cid == 0)
    def _():
        pltpu.sync_copy(acc_sh.at[pl.ds(rows0, nrows)],
                        num_hbm.at[pl.ds(rows0, nrows)])

    @pl.when(cid == 1)
    def _():
        pltpu.sync_copy(acc_sh.at[pl.ds(rows0, nrows)],
                        den_hbm.at[pl.ds(rows0, nrows)])


_scatter2 = pl.kernel(
    _scatter2_body,
    out_type=(
        jax.ShapeDtypeStruct((_N_PAD, _H), _f32),
        jax.ShapeDtypeStruct((_N_PAD, _H), _f32),
    ),
    mesh=_MESH,
    scratch_types=[
        pltpu.VMEM((2, _CH), jnp.int32),
        pltpu.VMEM((2, _CH, _H), _f32),
        pltpu.VMEM_SHARED((_N_PAD, _H), _f32),
    ] + [pltpu.SemaphoreType.DMA] * 6,
)


def kernel(h, e, edge_index, params):
    src = edge_index[0]
    dst = edge_index[1]
    r2 = lambda b: b.reshape(1, -1)

    h = _mm(h, params['emb_h'][0], r2(params['emb_h'][1]), blk=2000)
    e = _mm(e, params['emb_e'][0], r2(params['emb_e'][1]), blk=3200)
    zeros = jnp.zeros((_N_PAD, _H), _f32)

    for lp in params['layers']:
        ah, db, eh = _node_tf(h, lp)
        gm, gb = _gather_db_e(db, eh, src, dst)
        e_out, sig, npay = _edge_fuse(e, gm, gb, lp['C'][0], lp['C'][1])
        num, den = _scatter2(npay, sig, dst, zeros)
        h = _h_update(h, ah, num[:_N], den[:_N])
        e = e_out

    w1, b1 = params['mlp_e'][0]
    hn, p1, p2 = _node_ro(h, params['mlp_n'], w1[:_H], w1[_H:], b1)
    g = _gather_p1_p2(p1, p2, src, dst)
    ef = _edge_mlp(g, params['mlp_e'][1], params['mlp_e'][2])
    return hn, ef


# speedup vs baseline: 1.0055x; 1.0055x over previous
import functools

import jax
import jax.numpy as jnp
from jax import lax
from jax.experimental import pallas as pl
from jax.experimental.pallas import tpu as pltpu
from jax.experimental.pallas import tpu_sc as plsc

_N = 10000
_E = 320000
_H = 128
_NC = 2
_NS = 16
_NW = _NC * _NS
_CH = 80

_f32 = jnp.float32


def _dot(a, b):
    return jnp.dot(a, b, preferred_element_type=_f32)


def _mm_bias_kernel(x_ref, w_ref, b_ref, o_ref):
    o_ref[...] = _dot(x_ref[...], w_ref[...]) + b_ref[...]


def _mm(x, w, b, blk):
    m, k = x.shape
    n = w.shape[1]
    return pl.pallas_call(
        _mm_bias_kernel,
        grid=(m // blk,),
        in_specs=[
            pl.BlockSpec((blk, k), lambda i: (i, 0)),
            pl.BlockSpec((k, n), lambda i: (0, 0)),
            pl.BlockSpec((1, n), lambda i: (0, 0)),
        ],
        out_specs=pl.BlockSpec((blk, n), lambda i: (i, 0)),
        out_shape=jax.ShapeDtypeStruct((m, n), _f32),
    )(x, w, b)


def _node_tf_kernel(h_ref, wa, ba, wb, bb, wd, bd, we, be,
                    ah_ref, db_ref, eh_ref):
    h = h_ref[...]
    ah_ref[...] = _dot(h, wa[...]) + ba[...]
    db_ref[:, :_H] = _dot(h, wd[...]) + bd[...]
    db_ref[:, _H:] = _dot(h, wb[...]) + bb[...]
    eh_ref[...] = _dot(h, we[...]) + be[...]


def _node_tf(h, lp, blk=2000):
    wspec = pl.BlockSpec((_H, _H), lambda i: (0, 0))
    bspec = pl.BlockSpec((1, _H), lambda i: (0, 0))
    r2 = lambda b: b.reshape(1, _H)
    return pl.pallas_call(
        _node_tf_kernel,
        grid=(_N // blk,),
        in_specs=[pl.BlockSpec((blk, _H), lambda i: (i, 0))]
        + [wspec, bspec] * 4,
        out_specs=[
            pl.BlockSpec((blk, _H), lambda i: (i, 0)),
            pl.BlockSpec((blk, 2 * _H), lambda i: (i, 0)),
            pl.BlockSpec((blk, _H), lambda i: (i, 0)),
        ],
        out_shape=[
            jax.ShapeDtypeStruct((_N, _H), _f32),
            jax.ShapeDtypeStruct((_N, 2 * _H), _f32),
            jax.ShapeDtypeStruct((_N, _H), _f32),
        ],
    )(h, lp['A'][0], r2(lp['A'][1]), lp['B'][0], r2(lp['B'][1]),
      lp['D'][0], r2(lp['D'][1]), lp['E'][0], r2(lp['E'][1]))


def _edge_fuse_kernel(e_ref, gm_ref, gb_ref, wc, bc,
                      eo_ref, sig_ref, np_ref):
    ce = _dot(e_ref[...], wc[...]) + bc[...]
    en = ce + gm_ref[...]
    sig = jax.nn.sigmoid(en)
    eo_ref[...] = e_ref[...] + jnp.maximum(en, 0.0)
    sig_ref[...] = sig
    np_ref[...] = sig * gb_ref[...]


def _edge_fuse(e, gm, gb, wc, bc, blk=1280):
    espec = pl.BlockSpec((blk, _H), lambda i: (i, 0))
    return pl.pallas_call(
        _edge_fuse_kernel,
        grid=(_E // blk,),
        in_specs=[
            espec, espec, espec,
            pl.BlockSpec((_H, _H), lambda i: (0, 0)),
            pl.BlockSpec((1, _H), lambda i: (0, 0)),
        ],
        out_specs=[espec, espec, espec],
        out_shape=[jax.ShapeDtypeStruct((_E, _H), _f32)] * 3,
    )(e, gm, gb, wc, bc.reshape(1, _H))


def _h_update_kernel(h_ref, ah_ref, num_ref, den_ref, o_ref):
    o_ref[...] = h_ref[...] + jnp.maximum(
        ah_ref[...] + num_ref[...] / (den_ref[...] + 1e-6), 0.0)


def _h_update(h, ah, num, den, blk=2000):
    spec = pl.BlockSpec((blk, _H), lambda i: (i, 0))
    return pl.pallas_call(
        _h_update_kernel,
        grid=(_N // blk,),
        in_specs=[spec] * 4,
        out_specs=spec,
        out_shape=jax.ShapeDtypeStruct((_N, _H), _f32),
    )(h, ah, num, den)


def _node_ro_kernel(h_ref, w1, b1, w2, b2, w3, b3, wea, web, beb,
                    hn_ref, p1_ref, p2_ref):
    h = h_ref[...]
    t = jnp.maximum(_dot(h, w1[...]) + b1[...], 0.0)
    t = jnp.maximum(_dot(t, w2[...]) + b2[...], 0.0)
    hn_ref[...] = _dot(t, w3[...]) + b3[...]
    p1_ref[...] = _dot(h, wea[...])
    p2_ref[...] = _dot(h, web[...]) + beb[...]


def _node_ro(h, mlp_n, wea, web, beb, blk=2000):
    specs = []
    args = [h]
    for (w, b) in mlp_n:
        k, n = w.shape
        specs += [pl.BlockSpec((k, n), lambda i: (0, 0)),
                  pl.BlockSpec((1, n), lambda i: (0, 0))]
        args += [w, b.reshape(1, n)]
    specs += [pl.BlockSpec((_H, _H), lambda i: (0, 0))] * 2
    specs += [pl.BlockSpec((1, _H), lambda i: (0, 0))]
    args += [wea, web, beb.reshape(1, _H)]
    hspec = pl.BlockSpec((blk, _H), lambda i: (i, 0))
    return pl.pallas_call(
        _node_ro_kernel,
        grid=(_N // blk,),
        in_specs=[hspec] + specs,
        out_specs=[hspec, hspec, hspec],
        out_shape=[jax.ShapeDtypeStruct((_N, _H), _f32)] * 3,
    )(*args)


def _edge_mlp_kernel(g_ref, w2, b2, w3, b3, o_ref):
    t = jnp.maximum(_dot(g_ref[...], w2[...]) + b2[...], 0.0)
    o_ref[...] = _dot(t, w3[...]) + b3[...]


def _edge_mlp(g, l2, l3, blk=1280):
    w2, b2 = l2
    w3, b3 = l3
    espec = pl.BlockSpec((blk, _H), lambda i: (i, 0))
    return pl.pallas_call(
        _edge_mlp_kernel,
        grid=(_E // blk,),
        in_specs=[
            espec,
            pl.BlockSpec(w2.shape, lambda i: (0, 0)),
            pl.BlockSpec((1, w2.shape[1]), lambda i: (0, 0)),
            pl.BlockSpec(w3.shape, lambda i: (0, 0)),
            pl.BlockSpec((1, w3.shape[1]), lambda i: (0, 0)),
        ],
        out_specs=espec,
        out_shape=jax.ShapeDtypeStruct((_E, _H), _f32),
    )(g, w2, b2.reshape(1, -1), w3, b3.reshape(1, -1))


_MESH = plsc.VectorSubcoreMesh(core_axis_name="c", subcore_axis_name="s")


def _make_gather2(mode):
    d1 = 2 * _H if mode == 'layer' else _H
    per_w = _E // _NW
    n_chunks = per_w // _CH

    def body(t1, t2, i1_hbm, i2_hbm, *refs):
        if mode == 'layer':
            (o1, o2, i1_v, i2_v, r1_v, r2_v,
             sg0, sg1, so0, so1) = refs
        else:
            (o1, i1_v, i2_v, r1_v, r2_v,
             sg0, sg1, so0, so1) = refs
        wid = lax.axis_index("s") * _NC + lax.axis_index("c")
        base = wid * per_w
        sg = (sg0, sg1)
        so = (so0, so1)

        pltpu.sync_copy(i1_hbm.at[pl.ds(base, per_w)], i1_v)
        pltpu.sync_copy(i2_hbm.at[pl.ds(base, per_w)], i2_v)

        def gather_issue(i, b):
            loff = pl.multiple_of(i * _CH, 8)
            pltpu.async_copy(t1.at[i1_v.at[pl.ds(loff, _CH)]],
                             r1_v.at[b], sg[b])
            pltpu.async_copy(t2.at[i2_v.at[pl.ds(loff, _CH)]],
                             r2_v.at[b], sg[b])

        def gather_wait(i, b):
            loff = pl.multiple_of(i * _CH, 8)
            pltpu.make_async_copy(t1.at[i1_v.at[pl.ds(loff, _CH)]],
                                  r1_v.at[b], sg[b]).wait()
            pltpu.make_async_copy(t2.at[i2_v.at[pl.ds(loff, _CH)]],
                                  r2_v.at[b], sg[b]).wait()

        def combine(b):
            @pl.loop(0, _CH, unroll=4)
            def _(r):
                for j in range(_H // 16):
                    sl = pl.ds(j * 16, 16)
                    x = r1_v[b, r, sl] + r2_v[b, r, sl]
                    if mode == 'readout':
                        x = jnp.maximum(x, 0.0)
                    r2_v[b, r, sl] = x

        def write_issue(i, b):
            off = pl.multiple_of(base + i * _CH, 8)
            pltpu.async_copy(r2_v.at[b], o1.at[pl.ds(off, _CH)], so[b])
            if mode == 'layer':
                pltpu.async_copy(r1_v.at[b, slice(None), pl.ds(_H, _H)],
                                 o2.at[pl.ds(off, _CH)], so[b])

        def write_wait(i, b):
            off = pl.multiple_of(base + i * _CH, 8)
            pltpu.make_async_copy(
                r2_v.at[b], o1.at[pl.ds(off, _CH)], so[b]).wait()
            if mode == 'layer':
                pltpu.make_async_copy(
                    r1_v.at[b, slice(None), pl.ds(_H, _H)],
                    o2.at[pl.ds(off, _CH)], so[b]).wait()

        def maybe(cond, fn):
            if cond is True:
                fn()
            elif cond is not False:
                pl.when(cond)(fn)

        def stage(i, b, has_prev, has_prev2):
            maybe(has_prev2, lambda: write_wait(i - 2, b))
            gather_issue(i, b)

            def drain_prev():
                gather_wait(i - 1, 1 - b)
                combine(1 - b)
                write_issue(i - 1, 1 - b)
            maybe(has_prev, drain_prev)

        @pl.loop(0, n_chunks // 2)
        def _(j):
            i0 = j * 2
            stage(i0, 0, j > 0, j > 0)
            stage(i0 + 1, 1, True, j > 0)

        last = n_chunks - 1
        if n_chunks % 2 == 1:
            write_wait(last - 2, 0)
            gather_issue(last, 0)
            gather_wait(last - 1, 1)
            combine(1)
            write_issue(last - 1, 1)
            gather_wait(last, 0)
            combine(0)
            write_issue(last, 0)
            write_wait(last - 1, 1)
            write_wait(last, 0)
        else:
            gather_wait(last, 1)
            combine(1)
            write_issue(last, 1)
            write_wait(last - 1, 0)
            write_wait(last, 1)

    out_struct = jax.ShapeDtypeStruct((_E, _H), _f32)
    return pl.kernel(
        body,
        out_type=((out_struct, out_struct) if mode == 'layer'
                  else out_struct),
        mesh=_MESH,
        scratch_types=[
            pltpu.VMEM((per_w,), jnp.int32),
            pltpu.VMEM((per_w,), jnp.int32),
            pltpu.VMEM((2, _CH, d1), _f32),
            pltpu.VMEM((2, _CH, _H), _f32),
        ] + [pltpu.SemaphoreType.DMA] * 4,
    )


_gather_db_e = _make_gather2('layer')
_gather_p1_p2 = _make_gather2('readout')


_NROWS = 632
_N_PAD = _NROWS * _NS


def _scatter2_body(np_hbm, sig_hbm, dst_hbm, zero_hbm, num_hbm, den_hbm,
                   idx_v, pay_v, acc_sh, si0, si1, sp0, sp1, ss0, ss1):
    cid = lax.axis_index("c")
    sid = lax.axis_index("s")
    nrows = _NROWS
    rows0 = sid * nrows
    per_tile = _E // _NS
    ebase = sid * per_tile

    pltpu.sync_copy(zero_hbm.at[pl.ds(rows0, nrows)],
                    acc_sh.at[pl.ds(rows0, nrows)])
    plsc.subcore_barrier()

    n_chunks = per_tile // _CH

    def scatter_from(src_hbm):
        si = (si0, si1)
        sp = (sp0, sp1)
        ss = (ss0, ss1)

        def load(i, b):
            off = pl.multiple_of(ebase + i * _CH, 8)
            pltpu.async_copy(dst_hbm.at[pl.ds(off, _CH)], idx_v.at[b], si[b])
            pltpu.async_copy(src_hbm.at[pl.ds(off, _CH)], pay_v.at[b], sp[b])

        def load_wait(i, b):
            off = pl.multiple_of(ebase + i * _CH, 8)
            pltpu.make_async_copy(
                dst_hbm.at[pl.ds(off, _CH)], idx_v.at[b], si[b]).wait()
            pltpu.make_async_copy(
                src_hbm.at[pl.ds(off, _CH)], pay_v.at[b], sp[b]).wait()

        def scat_issue(b):
            pltpu.async_copy(pay_v.at[b], acc_sh.at[idx_v.at[b]], ss[b],
                             add=True)

        def scat_wait(b):
            pltpu.make_async_copy(pay_v.at[b], acc_sh.at[idx_v.at[b]],
                                  ss[b]).wait()

        def maybe(cond, fn):
            if cond is True:
                fn()
            elif cond is not False:
                pl.when(cond)(fn)

        def stage(i, b, has_prev, has_next):
            load_wait(i, b)
            scat_issue(b)
            maybe(has_prev, lambda: scat_wait(1 - b))
            maybe(has_next, lambda: load(i + 1, 1 - b))

        load(0, 0)

        @pl.loop(0, n_chunks // 2)
        def _(j):
            i0 = j * 2
            stage(i0, 0, j > 0, True)
            stage(i0 + 1, 1, True, i0 + 2 < n_chunks)

        scat_wait(1)

    @pl.when(cid == 0)
    def _():
        scatter_from(np_hbm)

    @pl.when(cid == 1)
    def _():
        scatter_from(sig_hbm)

    plsc.subcore_barrier()

    @pl.when(cid == 0)
    def _():
        pltpu.sync_copy(acc_sh.at[pl.ds(rows0, nrows)],
                        num_hbm.at[pl.ds(rows0, nrows)])

    @pl.when(cid == 1)
    def _():
        pltpu.sync_copy(acc_sh.at[pl.ds(rows0, nrows)],
                        den_hbm.at[pl.ds(rows0, nrows)])


_scatter2 = pl.kernel(
    _scatter2_body,
    out_type=(
        jax.ShapeDtypeStruct((_N_PAD, _H), _f32),
        jax.ShapeDtypeStruct((_N_PAD, _H), _f32),
    ),
    mesh=_MESH,
    scratch_types=[
        pltpu.VMEM((2, _CH), jnp.int32),
        pltpu.VMEM((2, _CH, _H), _f32),
        pltpu.VMEM_SHARED((_N_PAD, _H), _f32),
    ] + [pltpu.SemaphoreType.DMA] * 6,
)


def kernel(h, e, edge_index, params):
    src = edge_index[0]
    dst = edge_index[1]
    r2 = lambda b: b.reshape(1, -1)

    h = _mm(h, params['emb_h'][0], r2(params['emb_h'][1]), blk=2000)
    e = _mm(e, params['emb_e'][0], r2(params['emb_e'][1]), blk=3200)
    zeros = jnp.zeros((_N_PAD, _H), _f32)

    for lp in params['layers']:
        ah, db, eh = _node_tf(h, lp)
        gm, gb = _gather_db_e(db, eh, src, dst)
        e_out, sig, npay = _edge_fuse(e, gm, gb, lp['C'][0], lp['C'][1])
        num, den = _scatter2(npay, sig, dst, zeros)
        h = _h_update(h, ah, num[:_N], den[:_N])
        e = e_out

    w1, b1 = params['mlp_e'][0]
    hn, p1, p2 = _node_ro(h, params['mlp_n'], w1[:_H], w1[_H:], b1)
    g = _gather_p1_p2(p1, p2, src, dst)
    ef = _edge_mlp(g, params['mlp_e'][1], params['mlp_e'][2])
    return hn, ef


# speedup vs baseline: 1.0983x; 1.0923x over previous
import functools

import jax
import jax.numpy as jnp
from jax import lax
from jax.experimental import pallas as pl
from jax.experimental.pallas import tpu as pltpu
from jax.experimental.pallas import tpu_sc as plsc

_N = 10000
_E = 320000
_H = 128
_NC = 2
_NS = 16
_NW = _NC * _NS
_CH = 80

_f32 = jnp.float32


def _dot(a, b):
    return jnp.dot(a, b, preferred_element_type=_f32)


def _mm_bias_kernel(x_ref, w_ref, b_ref, o_ref):
    o_ref[...] = _dot(x_ref[...], w_ref[...]) + b_ref[...]


def _mm(x, w, b, blk):
    m, k = x.shape
    n = w.shape[1]
    return pl.pallas_call(
        _mm_bias_kernel,
        grid=(m // blk,),
        in_specs=[
            pl.BlockSpec((blk, k), lambda i: (i, 0)),
            pl.BlockSpec((k, n), lambda i: (0, 0)),
            pl.BlockSpec((1, n), lambda i: (0, 0)),
        ],
        out_specs=pl.BlockSpec((blk, n), lambda i: (i, 0)),
        out_shape=jax.ShapeDtypeStruct((m, n), _f32),
    )(x, w, b)


def _node_tf_kernel(h_ref, wa, ba, wb, bb, wd, bd, we, be,
                    ah_ref, db_ref, eh_ref):
    h = h_ref[...]
    ah_ref[...] = _dot(h, wa[...]) + ba[...]
    db_ref[:, :_H] = _dot(h, wd[...]) + bd[...]
    db_ref[:, _H:] = _dot(h, wb[...]) + bb[...]
    eh_ref[...] = _dot(h, we[...]) + be[...]


def _node_tf(h, lp, blk=2000):
    wspec = pl.BlockSpec((_H, _H), lambda i: (0, 0))
    bspec = pl.BlockSpec((1, _H), lambda i: (0, 0))
    r2 = lambda b: b.reshape(1, _H)
    return pl.pallas_call(
        _node_tf_kernel,
        grid=(_N // blk,),
        in_specs=[pl.BlockSpec((blk, _H), lambda i: (i, 0))]
        + [wspec, bspec] * 4,
        out_specs=[
            pl.BlockSpec((blk, _H), lambda i: (i, 0)),
            pl.BlockSpec((blk, 2 * _H), lambda i: (i, 0)),
            pl.BlockSpec((blk, _H), lambda i: (i, 0)),
        ],
        out_shape=[
            jax.ShapeDtypeStruct((_N, _H), _f32),
            jax.ShapeDtypeStruct((_N, 2 * _H), _f32),
            jax.ShapeDtypeStruct((_N, _H), _f32),
        ],
    )(h, lp['A'][0], r2(lp['A'][1]), lp['B'][0], r2(lp['B'][1]),
      lp['D'][0], r2(lp['D'][1]), lp['E'][0], r2(lp['E'][1]))


def _edge_fuse_kernel(e_ref, gdb_ref, ge_ref, wc, bc,
                      eo_ref, sig_ref, np_ref):
    ce = _dot(e_ref[...], wc[...]) + bc[...]
    en = ce + gdb_ref[:, :_H] + ge_ref[...]
    sig = jax.nn.sigmoid(en)
    eo_ref[...] = e_ref[...] + jnp.maximum(en, 0.0)
    sig_ref[...] = sig
    np_ref[...] = sig * gdb_ref[:, _H:]


def _edge_fuse(e, gdb, ge, wc, bc, blk=1280):
    espec = pl.BlockSpec((blk, _H), lambda i: (i, 0))
    return pl.pallas_call(
        _edge_fuse_kernel,
        grid=(_E // blk,),
        in_specs=[
            espec,
            pl.BlockSpec((blk, 2 * _H), lambda i: (i, 0)),
            espec,
            pl.BlockSpec((_H, _H), lambda i: (0, 0)),
            pl.BlockSpec((1, _H), lambda i: (0, 0)),
        ],
        out_specs=[espec, espec, espec],
        out_shape=[jax.ShapeDtypeStruct((_E, _H), _f32)] * 3,
    )(e, gdb, ge, wc, bc.reshape(1, _H))


def _h_update_kernel(h_ref, ah_ref, num_ref, den_ref, o_ref):
    o_ref[...] = h_ref[...] + jnp.maximum(
        ah_ref[...] + num_ref[...] / (den_ref[...] + 1e-6), 0.0)


def _h_update(h, ah, num, den, blk=2000):
    spec = pl.BlockSpec((blk, _H), lambda i: (i, 0))
    return pl.pallas_call(
        _h_update_kernel,
        grid=(_N // blk,),
        in_specs=[spec] * 4,
        out_specs=spec,
        out_shape=jax.ShapeDtypeStruct((_N, _H), _f32),
    )(h, ah, num, den)


def _node_ro_kernel(h_ref, w1, b1, w2, b2, w3, b3, wea, web, beb,
                    hn_ref, p1_ref, p2_ref):
    h = h_ref[...]
    t = jnp.maximum(_dot(h, w1[...]) + b1[...], 0.0)
    t = jnp.maximum(_dot(t, w2[...]) + b2[...], 0.0)
    hn_ref[...] = _dot(t, w3[...]) + b3[...]
    p1_ref[...] = _dot(h, wea[...])
    p2_ref[...] = _dot(h, web[...]) + beb[...]


def _node_ro(h, mlp_n, wea, web, beb, blk=2000):
    specs = []
    args = [h]
    for (w, b) in mlp_n:
        k, n = w.shape
        specs += [pl.BlockSpec((k, n), lambda i: (0, 0)),
                  pl.BlockSpec((1, n), lambda i: (0, 0))]
        args += [w, b.reshape(1, n)]
    specs += [pl.BlockSpec((_H, _H), lambda i: (0, 0))] * 2
    specs += [pl.BlockSpec((1, _H), lambda i: (0, 0))]
    args += [wea, web, beb.reshape(1, _H)]
    hspec = pl.BlockSpec((blk, _H), lambda i: (i, 0))
    return pl.pallas_call(
        _node_ro_kernel,
        grid=(_N // blk,),
        in_specs=[hspec] + specs,
        out_specs=[hspec, hspec, hspec],
        out_shape=[jax.ShapeDtypeStruct((_N, _H), _f32)] * 3,
    )(*args)


def _edge_mlp_kernel(g1_ref, g2_ref, w2, b2, w3, b3, o_ref):
    g = jnp.maximum(g1_ref[...] + g2_ref[...], 0.0)
    t = jnp.maximum(_dot(g, w2[...]) + b2[...], 0.0)
    o_ref[...] = _dot(t, w3[...]) + b3[...]


def _edge_mlp(g1, g2, l2, l3, blk=1280):
    w2, b2 = l2
    w3, b3 = l3
    espec = pl.BlockSpec((blk, _H), lambda i: (i, 0))
    return pl.pallas_call(
        _edge_mlp_kernel,
        grid=(_E // blk,),
        in_specs=[
            espec, espec,
            pl.BlockSpec(w2.shape, lambda i: (0, 0)),
            pl.BlockSpec((1, w2.shape[1]), lambda i: (0, 0)),
            pl.BlockSpec(w3.shape, lambda i: (0, 0)),
            pl.BlockSpec((1, w3.shape[1]), lambda i: (0, 0)),
        ],
        out_specs=espec,
        out_shape=jax.ShapeDtypeStruct((_E, _H), _f32),
    )(g1, g2, w2, b2.reshape(1, -1), w3, b3.reshape(1, -1))


_MESH = plsc.VectorSubcoreMesh(core_axis_name="c", subcore_axis_name="s")


def _make_gather2(d1, d2):
    per_w = _E // _NW
    n_chunks = per_w // _CH

    def body(t1, t2, i1_hbm, i2_hbm, o1, o2,
             i1_v, i2_v, r1_v, r2_v, sg0, sg1, so0, so1):
        wid = lax.axis_index("s") * _NC + lax.axis_index("c")
        base = wid * per_w
        sg = (sg0, sg1)
        so = (so0, so1)

        pltpu.sync_copy(i1_hbm.at[pl.ds(base, per_w)], i1_v)
        pltpu.sync_copy(i2_hbm.at[pl.ds(base, per_w)], i2_v)

        def gather_issue(i, b):
            loff = pl.multiple_of(i * _CH, 8)
            pltpu.async_copy(t1.at[i1_v.at[pl.ds(loff, _CH)]],
                             r1_v.at[b], sg[b])
            pltpu.async_copy(t2.at[i2_v.at[pl.ds(loff, _CH)]],
                             r2_v.at[b], sg[b])

        def gather_wait(i, b):
            loff = pl.multiple_of(i * _CH, 8)
            pltpu.make_async_copy(t1.at[i1_v.at[pl.ds(loff, _CH)]],
                                  r1_v.at[b], sg[b]).wait()
            pltpu.make_async_copy(t2.at[i2_v.at[pl.ds(loff, _CH)]],
                                  r2_v.at[b], sg[b]).wait()

        def write_issue(i, b):
            off = pl.multiple_of(base + i * _CH, 8)
            pltpu.async_copy(r1_v.at[b], o1.at[pl.ds(off, _CH)], so[b])
            pltpu.async_copy(r2_v.at[b], o2.at[pl.ds(off, _CH)], so[b])

        def write_wait(i, b):
            off = pl.multiple_of(base + i * _CH, 8)
            pltpu.make_async_copy(
                r1_v.at[b], o1.at[pl.ds(off, _CH)], so[b]).wait()
            pltpu.make_async_copy(
                r2_v.at[b], o2.at[pl.ds(off, _CH)], so[b]).wait()

        def maybe(cond, fn):
            if cond is True:
                fn()
            elif cond is not False:
                pl.when(cond)(fn)

        def stage(i, b, has_prev, has_prev2):
            maybe(has_prev2, lambda: write_wait(i - 2, b))
            gather_issue(i, b)

            def drain_prev():
                gather_wait(i - 1, 1 - b)
                write_issue(i - 1, 1 - b)
            maybe(has_prev, drain_prev)

        @pl.loop(0, n_chunks // 2)
        def _(j):
            i0 = j * 2
            stage(i0, 0, j > 0, j > 0)
            stage(i0 + 1, 1, True, j > 0)

        last = n_chunks - 1
        if n_chunks % 2 == 1:
            write_wait(last - 2, 0)
            gather_issue(last, 0)
            gather_wait(last - 1, 1)
            write_issue(last - 1, 1)
            gather_wait(last, 0)
            write_issue(last, 0)
            write_wait(last - 1, 1)
            write_wait(last, 0)
        else:
            gather_wait(last, 1)
            write_issue(last, 1)
            write_wait(last - 1, 0)
            write_wait(last, 1)

    return pl.kernel(
        body,
        out_type=(
            jax.ShapeDtypeStruct((_E, d1), _f32),
            jax.ShapeDtypeStruct((_E, d2), _f32),
        ),
        mesh=_MESH,
        scratch_types=[
            pltpu.VMEM((per_w,), jnp.int32),
            pltpu.VMEM((per_w,), jnp.int32),
            pltpu.VMEM((2, _CH, d1), _f32),
            pltpu.VMEM((2, _CH, d2), _f32),
        ] + [pltpu.SemaphoreType.DMA] * 4,
    )


_gather_db_e = _make_gather2(2 * _H, _H)
_gather_p1_p2 = _make_gather2(_H, _H)


_NROWS = 632
_N_PAD = _NROWS * _NS


def _scatter2_body(np_hbm, sig_hbm, dst_hbm, zero_hbm, num_hbm, den_hbm,
                   idx_v, pay_v, acc_sh, si0, si1, sp0, sp1, ss0, ss1):
    cid = lax.axis_index("c")
    sid = lax.axis_index("s")
    nrows = _NROWS
    rows0 = sid * nrows
    per_tile = _E // _NS
    ebase = sid * per_tile

    pltpu.sync_copy(zero_hbm.at[pl.ds(rows0, nrows)],
                    acc_sh.at[pl.ds(rows0, nrows)])
    plsc.subcore_barrier()

    n_chunks = per_tile // _CH

    def scatter_from(src_hbm):
        si = (si0, si1)
        sp = (sp0, sp1)
        ss = (ss0, ss1)

        def load(i, b):
            off = pl.multiple_of(ebase + i * _CH, 8)
            pltpu.async_copy(dst_hbm.at[pl.ds(off, _CH)], idx_v.at[b], si[b])
            pltpu.async_copy(src_hbm.at[pl.ds(off, _CH)], pay_v.at[b], sp[b])

        def load_wait(i, b):
            off = pl.multiple_of(ebase + i * _CH, 8)
            pltpu.make_async_copy(
                dst_hbm.at[pl.ds(off, _CH)], idx_v.at[b], si[b]).wait()
            pltpu.make_async_copy(
                src_hbm.at[pl.ds(off, _CH)], pay_v.at[b], sp[b]).wait()

        def scat_issue(b):
            pltpu.async_copy(pay_v.at[b], acc_sh.at[idx_v.at[b]], ss[b],
                             add=True)

        def scat_wait(b):
            pltpu.make_async_copy(pay_v.at[b], acc_sh.at[idx_v.at[b]],
                                  ss[b]).wait()

        def maybe(cond, fn):
            if cond is True:
                fn()
            elif cond is not False:
                pl.when(cond)(fn)

        def stage(i, b, has_prev, has_next):
            load_wait(i, b)
            scat_issue(b)
            maybe(has_prev, lambda: scat_wait(1 - b))
            maybe(has_next, lambda: load(i + 1, 1 - b))

        load(0, 0)

        @pl.loop(0, n_chunks // 2)
        def _(j):
            i0 = j * 2
            stage(i0, 0, j > 0, True)
            stage(i0 + 1, 1, True, i0 + 2 < n_chunks)

        scat_wait(1)

    @pl.when(cid == 0)
    def _():
        scatter_from(np_hbm)

    @pl.when(cid == 1)
    def _():
        scatter_from(sig_hbm)

    plsc.subcore_barrier()

    @pl.when(cid == 0)
    def _():
        pltpu.sync_copy(acc_sh.at[pl.ds(rows0, nrows)],
                        num_hbm.at[pl.ds(rows0, nrows)])

    @pl.when(cid == 1)
    def _():
        pltpu.sync_copy(acc_sh.at[pl.ds(rows0, nrows)],
                        den_hbm.at[pl.ds(rows0, nrows)])


_scatter2 = pl.kernel(
    _scatter2_body,
    out_type=(
        jax.ShapeDtypeStruct((_N_PAD, _H), _f32),
        jax.ShapeDtypeStruct((_N_PAD, _H), _f32),
    ),
    mesh=_MESH,
    scratch_types=[
        pltpu.VMEM((2, _CH), jnp.int32),
        pltpu.VMEM((2, _CH, _H), _f32),
        pltpu.VMEM_SHARED((_N_PAD, _H), _f32),
    ] + [pltpu.SemaphoreType.DMA] * 6,
)


def kernel(h, e, edge_index, params):
    src = edge_index[0]
    dst = edge_index[1]
    r2 = lambda b: b.reshape(1, -1)

    h = _mm(h, params['emb_h'][0], r2(params['emb_h'][1]), blk=2000)
    e = _mm(e, params['emb_e'][0], r2(params['emb_e'][1]), blk=3200)
    zeros = jnp.zeros((_N_PAD, _H), _f32)

    for lp in params['layers']:
        ah, db, eh = _node_tf(h, lp)
        gdb, ge = _gather_db_e(db, eh, src, dst)
        e_out, sig, npay = _edge_fuse(e, gdb, ge, lp['C'][0], lp['C'][1])
        num, den = _scatter2(npay, sig, dst, zeros)
        h = _h_update(h, ah, num[:_N], den[:_N])
        e = e_out

    w1, b1 = params['mlp_e'][0]
    hn, p1, p2 = _node_ro(h, params['mlp_n'], w1[:_H], w1[_H:], b1)
    g1, g2 = _gather_p1_p2(p1, p2, src, dst)
    ef = _edge_mlp(g1, g2, params['mlp_e'][1], params['mlp_e'][2])
    return hn, ef


# speedup vs baseline: 1.1640x; 1.0598x over previous
import functools

import jax
import jax.numpy as jnp
from jax import lax
from jax.experimental import pallas as pl
from jax.experimental.pallas import tpu as pltpu
from jax.experimental.pallas import tpu_sc as plsc

_N = 10000
_E = 320000
_H = 128
_NC = 2
_NS = 16
_NW = _NC * _NS
_CH = 80

_f32 = jnp.float32


def _dot(a, b):
    return jnp.dot(a, b, preferred_element_type=_f32)


def _mm_bias_kernel(x_ref, w_ref, b_ref, o_ref):
    o_ref[...] = _dot(x_ref[...], w_ref[...]) + b_ref[...]


def _mm(x, w, b, blk):
    m, k = x.shape
    n = w.shape[1]
    return pl.pallas_call(
        _mm_bias_kernel,
        grid=(m // blk,),
        in_specs=[
            pl.BlockSpec((blk, k), lambda i: (i, 0)),
            pl.BlockSpec((k, n), lambda i: (0, 0)),
            pl.BlockSpec((1, n), lambda i: (0, 0)),
        ],
        out_specs=pl.BlockSpec((blk, n), lambda i: (i, 0)),
        out_shape=jax.ShapeDtypeStruct((m, n), _f32),
    )(x, w, b)


def _node_tf_kernel(h_ref, wa, ba, wb, bb, wd, bd, we, be,
                    ah_ref, db_ref, eh_ref):
    h = h_ref[...]
    ah_ref[...] = _dot(h, wa[...]) + ba[...]
    db_ref[:, :_H] = _dot(h, wd[...]) + bd[...]
    db_ref[:, _H:] = _dot(h, wb[...]) + bb[...]
    eh_ref[...] = _dot(h, we[...]) + be[...]


def _node_tf(h, lp, blk=2000):
    wspec = pl.BlockSpec((_H, _H), lambda i: (0, 0))
    bspec = pl.BlockSpec((1, _H), lambda i: (0, 0))
    r2 = lambda b: b.reshape(1, _H)
    return pl.pallas_call(
        _node_tf_kernel,
        grid=(_N // blk,),
        in_specs=[pl.BlockSpec((blk, _H), lambda i: (i, 0))]
        + [wspec, bspec] * 4,
        out_specs=[
            pl.BlockSpec((blk, _H), lambda i: (i, 0)),
            pl.BlockSpec((blk, 2 * _H), lambda i: (i, 0)),
            pl.BlockSpec((blk, _H), lambda i: (i, 0)),
        ],
        out_shape=[
            jax.ShapeDtypeStruct((_N, _H), _f32),
            jax.ShapeDtypeStruct((_N, 2 * _H), _f32),
            jax.ShapeDtypeStruct((_N, _H), _f32),
        ],
    )(h, lp['A'][0], r2(lp['A'][1]), lp['B'][0], r2(lp['B'][1]),
      lp['D'][0], r2(lp['D'][1]), lp['E'][0], r2(lp['E'][1]))


def _edge_fuse1_kernel(z_ref, gdb_ref, ge_ref, wz, bp,
                       q_ref, sig_ref, np_ref):
    blk = gdb_ref.shape[0]
    ce = _dot(z_ref[...], wz[...]).reshape(blk, _H) + bp[...]
    en = ce + gdb_ref[:, :_H] + ge_ref[...]
    sig = jax.nn.sigmoid(en)
    q_ref[...] = jnp.maximum(en, 0.0)
    sig_ref[...] = sig
    np_ref[...] = sig * gdb_ref[:, _H:]


def _edge_fuse1(z, gdb, ge, wz, bp, blk=1280):
    espec = pl.BlockSpec((blk, _H), lambda i: (i, 0))
    return pl.pallas_call(
        _edge_fuse1_kernel,
        grid=(_E // blk,),
        in_specs=[
            pl.BlockSpec((blk // 8, _H), lambda i: (i, 0)),
            pl.BlockSpec((blk, 2 * _H), lambda i: (i, 0)),
            espec,
            pl.BlockSpec((_H, 8 * _H), lambda i: (0, 0)),
            pl.BlockSpec((1, _H), lambda i: (0, 0)),
        ],
        out_specs=[espec, espec, espec],
        out_shape=[jax.ShapeDtypeStruct((_E, _H), _f32)] * 3,
    )(z, gdb, ge, wz, bp.reshape(1, _H))


def _edge_fuse2_kernel(z_ref, q_ref, gdb_ref, ge_ref, wz, wc, bp,
                       sig_ref, np_ref):
    blk = gdb_ref.shape[0]
    ce = _dot(z_ref[...], wz[...]).reshape(blk, _H) + bp[...]
    ce = ce + _dot(q_ref[...], wc[...])
    en = ce + gdb_ref[:, :_H] + ge_ref[...]
    sig = jax.nn.sigmoid(en)
    sig_ref[...] = sig
    np_ref[...] = sig * gdb_ref[:, _H:]


def _edge_fuse2(z, q, gdb, ge, wz, wc, bp, blk=1280):
    espec = pl.BlockSpec((blk, _H), lambda i: (i, 0))
    return pl.pallas_call(
        _edge_fuse2_kernel,
        grid=(_E // blk,),
        in_specs=[
            pl.BlockSpec((blk // 8, _H), lambda i: (i, 0)),
            espec,
            pl.BlockSpec((blk, 2 * _H), lambda i: (i, 0)),
            espec,
            pl.BlockSpec((_H, 8 * _H), lambda i: (0, 0)),
            pl.BlockSpec((_H, _H), lambda i: (0, 0)),
            pl.BlockSpec((1, _H), lambda i: (0, 0)),
        ],
        out_specs=[espec, espec],
        out_shape=[jax.ShapeDtypeStruct((_E, _H), _f32)] * 2,
    )(z, q, gdb, ge, wz, wc, bp.reshape(1, _H))


def _h_update_kernel(h_ref, ah_ref, num_ref, den_ref, o_ref):
    o_ref[...] = h_ref[...] + jnp.maximum(
        ah_ref[...] + num_ref[...] / (den_ref[...] + 1e-6), 0.0)


def _h_update(h, ah, num, den, blk=2000):
    spec = pl.BlockSpec((blk, _H), lambda i: (i, 0))
    return pl.pallas_call(
        _h_update_kernel,
        grid=(_N // blk,),
        in_specs=[spec] * 4,
        out_specs=spec,
        out_shape=jax.ShapeDtypeStruct((_N, _H), _f32),
    )(h, ah, num, den)


def _node_ro_kernel(h_ref, w1, b1, w2, b2, w3, b3, wea, web, beb,
                    hn_ref, p1_ref, p2_ref):
    h = h_ref[...]
    t = jnp.maximum(_dot(h, w1[...]) + b1[...], 0.0)
    t = jnp.maximum(_dot(t, w2[...]) + b2[...], 0.0)
    hn_ref[...] = _dot(t, w3[...]) + b3[...]
    p1_ref[...] = _dot(h, wea[...])
    p2_ref[...] = _dot(h, web[...]) + beb[...]


def _node_ro(h, mlp_n, wea, web, beb, blk=2000):
    specs = []
    args = [h]
    for (w, b) in mlp_n:
        k, n = w.shape
        specs += [pl.BlockSpec((k, n), lambda i: (0, 0)),
                  pl.BlockSpec((1, n), lambda i: (0, 0))]
        args += [w, b.reshape(1, n)]
    specs += [pl.BlockSpec((_H, _H), lambda i: (0, 0))] * 2
    specs += [pl.BlockSpec((1, _H), lambda i: (0, 0))]
    args += [wea, web, beb.reshape(1, _H)]
    hspec = pl.BlockSpec((blk, _H), lambda i: (i, 0))
    return pl.pallas_call(
        _node_ro_kernel,
        grid=(_N // blk,),
        in_specs=[hspec] + specs,
        out_specs=[hspec, hspec, hspec],
        out_shape=[jax.ShapeDtypeStruct((_N, _H), _f32)] * 3,
    )(*args)


def _edge_mlp_kernel(g1_ref, g2_ref, w2, b2, w3, b3, o_ref):
    g = jnp.maximum(g1_ref[...] + g2_ref[...], 0.0)
    t = jnp.maximum(_dot(g, w2[...]) + b2[...], 0.0)
    o_ref[...] = _dot(t, w3[...]) + b3[...]


def _edge_mlp(g1, g2, l2, l3, blk=1280):
    w2, b2 = l2
    w3, b3 = l3
    espec = pl.BlockSpec((blk, _H), lambda i: (i, 0))
    return pl.pallas_call(
        _edge_mlp_kernel,
        grid=(_E // blk,),
        in_specs=[
            espec, espec,
            pl.BlockSpec(w2.shape, lambda i: (0, 0)),
            pl.BlockSpec((1, w2.shape[1]), lambda i: (0, 0)),
            pl.BlockSpec(w3.shape, lambda i: (0, 0)),
            pl.BlockSpec((1, w3.shape[1]), lambda i: (0, 0)),
        ],
        out_specs=espec,
        out_shape=jax.ShapeDtypeStruct((_E, _H), _f32),
    )(g1, g2, w2, b2.reshape(1, -1), w3, b3.reshape(1, -1))


_MESH = plsc.VectorSubcoreMesh(core_axis_name="c", subcore_axis_name="s")


def _make_gather2(d1, d2):
    per_w = _E // _NW
    n_chunks = per_w // _CH

    def body(t1, t2, i1_hbm, i2_hbm, o1, o2,
             i1_v, i2_v, r1_v, r2_v, sg0, sg1, so0, so1):
        wid = lax.axis_index("s") * _NC + lax.axis_index("c")
        base = wid * per_w
        sg = (sg0, sg1)
        so = (so0, so1)

        pltpu.sync_copy(i1_hbm.at[pl.ds(base, per_w)], i1_v)
        pltpu.sync_copy(i2_hbm.at[pl.ds(base, per_w)], i2_v)

        def gather_issue(i, b):
            loff = pl.multiple_of(i * _CH, 8)
            pltpu.async_copy(t1.at[i1_v.at[pl.ds(loff, _CH)]],
                             r1_v.at[b], sg[b])
            pltpu.async_copy(t2.at[i2_v.at[pl.ds(loff, _CH)]],
                             r2_v.at[b], sg[b])

        def gather_wait(i, b):
            loff = pl.multiple_of(i * _CH, 8)
            pltpu.make_async_copy(t1.at[i1_v.at[pl.ds(loff, _CH)]],
                                  r1_v.at[b], sg[b]).wait()
            pltpu.make_async_copy(t2.at[i2_v.at[pl.ds(loff, _CH)]],
                                  r2_v.at[b], sg[b]).wait()

        def write_issue(i, b):
            off = pl.multiple_of(base + i * _CH, 8)
            pltpu.async_copy(r1_v.at[b], o1.at[pl.ds(off, _CH)], so[b])
            pltpu.async_copy(r2_v.at[b], o2.at[pl.ds(off, _CH)], so[b])

        def write_wait(i, b):
            off = pl.multiple_of(base + i * _CH, 8)
            pltpu.make_async_copy(
                r1_v.at[b], o1.at[pl.ds(off, _CH)], so[b]).wait()
            pltpu.make_async_copy(
                r2_v.at[b], o2.at[pl.ds(off, _CH)], so[b]).wait()

        def maybe(cond, fn):
            if cond is True:
                fn()
            elif cond is not False:
                pl.when(cond)(fn)

        def stage(i, b, has_prev, has_prev2):
            maybe(has_prev2, lambda: write_wait(i - 2, b))
            gather_issue(i, b)

            def drain_prev():
                gather_wait(i - 1, 1 - b)
                write_issue(i - 1, 1 - b)
            maybe(has_prev, drain_prev)

        @pl.loop(0, n_chunks // 2)
        def _(j):
            i0 = j * 2
            stage(i0, 0, j > 0, j > 0)
            stage(i0 + 1, 1, True, j > 0)

        last = n_chunks - 1
        if n_chunks % 2 == 1:
            write_wait(last - 2, 0)
            gather_issue(last, 0)
            gather_wait(last - 1, 1)
            write_issue(last - 1, 1)
            gather_wait(last, 0)
            write_issue(last, 0)
            write_wait(last - 1, 1)
            write_wait(last, 0)
        else:
            gather_wait(last, 1)
            write_issue(last, 1)
            write_wait(last - 1, 0)
            write_wait(last, 1)

    return pl.kernel(
        body,
        out_type=(
            jax.ShapeDtypeStruct((_E, d1), _f32),
            jax.ShapeDtypeStruct((_E, d2), _f32),
        ),
        mesh=_MESH,
        scratch_types=[
            pltpu.VMEM((per_w,), jnp.int32),
            pltpu.VMEM((per_w,), jnp.int32),
            pltpu.VMEM((2, _CH, d1), _f32),
            pltpu.VMEM((2, _CH, d2), _f32),
        ] + [pltpu.SemaphoreType.DMA] * 4,
    )


_gather_db_e = _make_gather2(2 * _H, _H)
_gather_p1_p2 = _make_gather2(_H, _H)


_NROWS = 632
_N_PAD = _NROWS * _NS


def _scatter2_body(np_hbm, sig_hbm, dst_hbm, zero_hbm, num_hbm, den_hbm,
                   idx_v, pay_v, acc_sh, si0, si1, sp0, sp1, ss0, ss1):
    cid = lax.axis_index("c")
    sid = lax.axis_index("s")
    nrows = _NROWS
    rows0 = sid * nrows
    per_tile = _E // _NS
    ebase = sid * per_tile

    pltpu.sync_copy(zero_hbm.at[pl.ds(rows0, nrows)],
                    acc_sh.at[pl.ds(rows0, nrows)])
    plsc.subcore_barrier()

    n_chunks = per_tile // _CH

    def scatter_from(src_hbm):
        si = (si0, si1)
        sp = (sp0, sp1)
        ss = (ss0, ss1)

        def load(i, b):
            off = pl.multiple_of(ebase + i * _CH, 8)
            pltpu.async_copy(dst_hbm.at[pl.ds(off, _CH)], idx_v.at[b], si[b])
            pltpu.async_copy(src_hbm.at[pl.ds(off, _CH)], pay_v.at[b], sp[b])

        def load_wait(i, b):
            off = pl.multiple_of(ebase + i * _CH, 8)
            pltpu.make_async_copy(
                dst_hbm.at[pl.ds(off, _CH)], idx_v.at[b], si[b]).wait()
            pltpu.make_async_copy(
                src_hbm.at[pl.ds(off, _CH)], pay_v.at[b], sp[b]).wait()

        def scat_issue(b):
            pltpu.async_copy(pay_v.at[b], acc_sh.at[idx_v.at[b]], ss[b],
                             add=True)

        def scat_wait(b):
            pltpu.make_async_copy(pay_v.at[b], acc_sh.at[idx_v.at[b]],
                                  ss[b]).wait()

        def maybe(cond, fn):
            if cond is True:
                fn()
            elif cond is not False:
                pl.when(cond)(fn)

        def stage(i, b, has_prev, has_next):
            load_wait(i, b)
            scat_issue(b)
            maybe(has_prev, lambda: scat_wait(1 - b))
            maybe(has_next, lambda: load(i + 1, 1 - b))

        load(0, 0)

        @pl.loop(0, n_chunks // 2)
        def _(j):
            i0 = j * 2
            stage(i0, 0, j > 0, True)
            stage(i0 + 1, 1, True, i0 + 2 < n_chunks)

        scat_wait(1)

    @pl.when(cid == 0)
    def _():
        scatter_from(np_hbm)

    @pl.when(cid == 1)
    def _():
        scatter_from(sig_hbm)

    plsc.subcore_barrier()

    @pl.when(cid == 0)
    def _():
        pltpu.sync_copy(acc_sh.at[pl.ds(rows0, nrows)],
                        num_hbm.at[pl.ds(rows0, nrows)])

    @pl.when(cid == 1)
    def _():
        pltpu.sync_copy(acc_sh.at[pl.ds(rows0, nrows)],
                        den_hbm.at[pl.ds(rows0, nrows)])


_scatter2 = pl.kernel(
    _scatter2_body,
    out_type=(
        jax.ShapeDtypeStruct((_N_PAD, _H), _f32),
        jax.ShapeDtypeStruct((_N_PAD, _H), _f32),
    ),
    mesh=_MESH,
    scratch_types=[
        pltpu.VMEM((2, _CH), jnp.int32),
        pltpu.VMEM((2, _CH, _H), _f32),
        pltpu.VMEM_SHARED((_N_PAD, _H), _f32),
    ] + [pltpu.SemaphoreType.DMA] * 6,
)


def kernel(h, e, edge_index, params):
    src = edge_index[0]
    dst = edge_index[1]
    r2 = lambda b: b.reshape(1, -1)

    h = _mm(h, params['emb_h'][0], r2(params['emb_h'][1]), blk=2000)
    zeros = jnp.zeros((_N_PAD, _H), _f32)

    we, be = params['emb_e']
    z = e.reshape(_E // 8, _H)
    eye8 = jnp.eye(8, dtype=_f32)

    q = None
    for lp in params['layers']:
        wc, bc = lp['C']
        wz = jnp.kron(eye8, we @ wc)
        bp = be @ wc + bc
        ah, db, eh = _node_tf(h, lp)
        gdb, ge = _gather_db_e(db, eh, src, dst)
        if q is None:
            q, sig, npay = _edge_fuse1(z, gdb, ge, wz, bp)
        else:
            sig, npay = _edge_fuse2(z, q, gdb, ge, wz, wc, bp)
        num, den = _scatter2(npay, sig, dst, zeros)
        h = _h_update(h, ah, num, den)

    w1, b1 = params['mlp_e'][0]
    hn, p1, p2 = _node_ro(h, params['mlp_n'], w1[:_H], w1[_H:], b1)
    g1, g2 = _gather_p1_p2(p1, p2, src, dst)
    ef = _edge_mlp(g1, g2, params['mlp_e'][1], params['mlp_e'][2])
    return hn, ef


# speedup vs baseline: 1.2706x; 1.0916x over previous
import functools

import jax
import jax.numpy as jnp
from jax import lax
from jax.experimental import pallas as pl
from jax.experimental.pallas import tpu as pltpu
from jax.experimental.pallas import tpu_sc as plsc

_N = 10000
_E = 320000
_H = 128
_NC = 2
_NS = 16
_NW = _NC * _NS
_CH = 80

_f32 = jnp.float32


def _dot(a, b):
    return jnp.dot(a, b, preferred_element_type=_f32)


def _mm_bias_kernel(x_ref, w_ref, b_ref, o_ref):
    o_ref[...] = _dot(x_ref[...], w_ref[...]) + b_ref[...]


def _mm(x, w, b, blk):
    m, k = x.shape
    n = w.shape[1]
    return pl.pallas_call(
        _mm_bias_kernel,
        grid=(m // blk,),
        in_specs=[
            pl.BlockSpec((blk, k), lambda i: (i, 0)),
            pl.BlockSpec((k, n), lambda i: (0, 0)),
            pl.BlockSpec((1, n), lambda i: (0, 0)),
        ],
        out_specs=pl.BlockSpec((blk, n), lambda i: (i, 0)),
        out_shape=jax.ShapeDtypeStruct((m, n), _f32),
    )(x, w, b)


def _node_tf_kernel(h_ref, wa, ba, wb, bb, wd, bd, we, be,
                    ah_ref, db_ref, eh_ref):
    h = h_ref[...]
    ah_ref[...] = _dot(h, wa[...]) + ba[...]
    db_ref[:, :_H] = _dot(h, wd[...]) + bd[...]
    db_ref[:, _H:] = _dot(h, wb[...]) + bb[...]
    eh_ref[...] = _dot(h, we[...]) + be[...]


def _node_tf(h, lp, blk=2000):
    wspec = pl.BlockSpec((_H, _H), lambda i: (0, 0))
    bspec = pl.BlockSpec((1, _H), lambda i: (0, 0))
    r2 = lambda b: b.reshape(1, _H)
    return pl.pallas_call(
        _node_tf_kernel,
        grid=(_N // blk,),
        in_specs=[pl.BlockSpec((blk, _H), lambda i: (i, 0))]
        + [wspec, bspec] * 4,
        out_specs=[
            pl.BlockSpec((blk, _H), lambda i: (i, 0)),
            pl.BlockSpec((blk, 2 * _H), lambda i: (i, 0)),
            pl.BlockSpec((blk, _H), lambda i: (i, 0)),
        ],
        out_shape=[
            jax.ShapeDtypeStruct((_N, _H), _f32),
            jax.ShapeDtypeStruct((_N, 2 * _H), _f32),
            jax.ShapeDtypeStruct((_N, _H), _f32),
        ],
    )(h, lp['A'][0], r2(lp['A'][1]), lp['B'][0], r2(lp['B'][1]),
      lp['D'][0], r2(lp['D'][1]), lp['E'][0], r2(lp['E'][1]))


def _edge_fuse1_kernel(z_ref, gdb_ref, ge_ref, wz, bp,
                       q_ref, sig_ref, np_ref):
    blk = gdb_ref.shape[0]
    ce = _dot(z_ref[...], wz[...]).reshape(blk, _H) + bp[...]
    en = ce + gdb_ref[:, :_H] + ge_ref[...]
    sig = jax.nn.sigmoid(en)
    q_ref[...] = jnp.maximum(en, 0.0)
    sig_ref[...] = sig
    np_ref[...] = sig * gdb_ref[:, _H:]


def _edge_fuse1(z, gdb, ge, wz, bp, e_part, zoff, blk=1280):
    espec = pl.BlockSpec((blk, _H), lambda i: (i, 0))
    return pl.pallas_call(
        _edge_fuse1_kernel,
        grid=(e_part // blk,),
        in_specs=[
            pl.BlockSpec((blk // 8, _H), lambda i, z0=zoff: (i + z0, 0)),
            pl.BlockSpec((blk, 2 * _H), lambda i: (i, 0)),
            espec,
            pl.BlockSpec((_H, 8 * _H), lambda i: (0, 0)),
            pl.BlockSpec((1, _H), lambda i: (0, 0)),
        ],
        out_specs=[espec, espec, espec],
        out_shape=[jax.ShapeDtypeStruct((e_part, _H), _f32)] * 3,
    )(z, gdb, ge, wz, bp.reshape(1, _H))


def _edge_fuse2_kernel(z_ref, q_ref, gdb_ref, ge_ref, wz, wc, bp,
                       sig_ref, np_ref):
    blk = gdb_ref.shape[0]
    ce = _dot(z_ref[...], wz[...]).reshape(blk, _H) + bp[...]
    ce = ce + _dot(q_ref[...], wc[...])
    en = ce + gdb_ref[:, :_H] + ge_ref[...]
    sig = jax.nn.sigmoid(en)
    sig_ref[...] = sig
    np_ref[...] = sig * gdb_ref[:, _H:]


def _edge_fuse2(z, q, gdb, ge, wz, wc, bp, e_part, zoff, blk=1280):
    espec = pl.BlockSpec((blk, _H), lambda i: (i, 0))
    return pl.pallas_call(
        _edge_fuse2_kernel,
        grid=(e_part // blk,),
        in_specs=[
            pl.BlockSpec((blk // 8, _H), lambda i, z0=zoff: (i + z0, 0)),
            espec,
            pl.BlockSpec((blk, 2 * _H), lambda i: (i, 0)),
            espec,
            pl.BlockSpec((_H, 8 * _H), lambda i: (0, 0)),
            pl.BlockSpec((_H, _H), lambda i: (0, 0)),
            pl.BlockSpec((1, _H), lambda i: (0, 0)),
        ],
        out_specs=[espec, espec],
        out_shape=[jax.ShapeDtypeStruct((e_part, _H), _f32)] * 2,
    )(z, q, gdb, ge, wz, wc, bp.reshape(1, _H))


def _h_update_kernel(h_ref, ah_ref, na_ref, da_ref, nb_ref, db_ref, o_ref):
    num = na_ref[...] + nb_ref[...]
    den = da_ref[...] + db_ref[...]
    o_ref[...] = h_ref[...] + jnp.maximum(
        ah_ref[...] + num / (den + 1e-6), 0.0)


def _h_update(h, ah, na, da, nb, db, blk=2000):
    spec = pl.BlockSpec((blk, _H), lambda i: (i, 0))
    return pl.pallas_call(
        _h_update_kernel,
        grid=(_N // blk,),
        in_specs=[spec] * 6,
        out_specs=spec,
        out_shape=jax.ShapeDtypeStruct((_N, _H), _f32),
    )(h, ah, na, da, nb, db)


def _node_ro_kernel(h_ref, w1, b1, w2, b2, w3, b3, wea, web, beb,
                    hn_ref, p1_ref, p2_ref):
    h = h_ref[...]
    t = jnp.maximum(_dot(h, w1[...]) + b1[...], 0.0)
    t = jnp.maximum(_dot(t, w2[...]) + b2[...], 0.0)
    hn_ref[...] = _dot(t, w3[...]) + b3[...]
    p1_ref[...] = _dot(h, wea[...])
    p2_ref[...] = _dot(h, web[...]) + beb[...]


def _node_ro(h, mlp_n, wea, web, beb, blk=2000):
    specs = []
    args = [h]
    for (w, b) in mlp_n:
        k, n = w.shape
        specs += [pl.BlockSpec((k, n), lambda i: (0, 0)),
                  pl.BlockSpec((1, n), lambda i: (0, 0))]
        args += [w, b.reshape(1, n)]
    specs += [pl.BlockSpec((_H, _H), lambda i: (0, 0))] * 2
    specs += [pl.BlockSpec((1, _H), lambda i: (0, 0))]
    args += [wea, web, beb.reshape(1, _H)]
    hspec = pl.BlockSpec((blk, _H), lambda i: (i, 0))
    return pl.pallas_call(
        _node_ro_kernel,
        grid=(_N // blk,),
        in_specs=[hspec] + specs,
        out_specs=[hspec, hspec, hspec],
        out_shape=[jax.ShapeDtypeStruct((_N, _H), _f32)] * 3,
    )(*args)


def _edge_mlp_kernel(g1_ref, g2_ref, w2, b2, w3, b3, o_ref):
    g = jnp.maximum(g1_ref[...] + g2_ref[...], 0.0)
    t = jnp.maximum(_dot(g, w2[...]) + b2[...], 0.0)
    o_ref[...] = _dot(t, w3[...]) + b3[...]


def _edge_mlp(g1, g2, l2, l3, blk=1280):
    w2, b2 = l2
    w3, b3 = l3
    espec = pl.BlockSpec((blk, _H), lambda i: (i, 0))
    return pl.pallas_call(
        _edge_mlp_kernel,
        grid=(_E // blk,),
        in_specs=[
            espec, espec,
            pl.BlockSpec(w2.shape, lambda i: (0, 0)),
            pl.BlockSpec((1, w2.shape[1]), lambda i: (0, 0)),
            pl.BlockSpec(w3.shape, lambda i: (0, 0)),
            pl.BlockSpec((1, w3.shape[1]), lambda i: (0, 0)),
        ],
        out_specs=espec,
        out_shape=jax.ShapeDtypeStruct((_E, _H), _f32),
    )(g1, g2, w2, b2.reshape(1, -1), w3, b3.reshape(1, -1))


_MESH = plsc.VectorSubcoreMesh(core_axis_name="c", subcore_axis_name="s")


def _make_gather2(d1, d2, e_part=_E):
    per_w = e_part // _NW
    n_chunks = per_w // _CH

    def body(t1, t2, i1_hbm, i2_hbm, o1, o2,
             i1_v, i2_v, r1_v, r2_v, sg0, sg1, so0, so1):
        wid = lax.axis_index("s") * _NC + lax.axis_index("c")
        base = wid * per_w
        sg = (sg0, sg1)
        so = (so0, so1)

        pltpu.sync_copy(i1_hbm.at[pl.ds(base, per_w)], i1_v)
        pltpu.sync_copy(i2_hbm.at[pl.ds(base, per_w)], i2_v)

        def gather_issue(i, b):
            loff = pl.multiple_of(i * _CH, 8)
            pltpu.async_copy(t1.at[i1_v.at[pl.ds(loff, _CH)]],
                             r1_v.at[b], sg[b])
            pltpu.async_copy(t2.at[i2_v.at[pl.ds(loff, _CH)]],
                             r2_v.at[b], sg[b])

        def gather_wait(i, b):
            loff = pl.multiple_of(i * _CH, 8)
            pltpu.make_async_copy(t1.at[i1_v.at[pl.ds(loff, _CH)]],
                                  r1_v.at[b], sg[b]).wait()
            pltpu.make_async_copy(t2.at[i2_v.at[pl.ds(loff, _CH)]],
                                  r2_v.at[b], sg[b]).wait()

        def write_issue(i, b):
            off = pl.multiple_of(base + i * _CH, 8)
            pltpu.async_copy(r1_v.at[b], o1.at[pl.ds(off, _CH)], so[b])
            pltpu.async_copy(r2_v.at[b], o2.at[pl.ds(off, _CH)], so[b])

        def write_wait(i, b):
            off = pl.multiple_of(base + i * _CH, 8)
            pltpu.make_async_copy(
                r1_v.at[b], o1.at[pl.ds(off, _CH)], so[b]).wait()
            pltpu.make_async_copy(
                r2_v.at[b], o2.at[pl.ds(off, _CH)], so[b]).wait()

        def maybe(cond, fn):
            if cond is True:
                fn()
            elif cond is not False:
                pl.when(cond)(fn)

        def stage(i, b, has_prev, has_prev2):
            maybe(has_prev2, lambda: write_wait(i - 2, b))
            gather_issue(i, b)

            def drain_prev():
                gather_wait(i - 1, 1 - b)
                write_issue(i - 1, 1 - b)
            maybe(has_prev, drain_prev)

        @pl.loop(0, n_chunks // 2)
        def _(j):
            i0 = j * 2
            stage(i0, 0, j > 0, j > 0)
            stage(i0 + 1, 1, True, j > 0)

        last = n_chunks - 1
        if n_chunks % 2 == 1:
            write_wait(last - 2, 0)
            gather_issue(last, 0)
            gather_wait(last - 1, 1)
            write_issue(last - 1, 1)
            gather_wait(last, 0)
            write_issue(last, 0)
            write_wait(last - 1, 1)
            write_wait(last, 0)
        else:
            gather_wait(last, 1)
            write_issue(last, 1)
            write_wait(last - 1, 0)
            write_wait(last, 1)

    return pl.kernel(
        body,
        out_type=(
            jax.ShapeDtypeStruct((e_part, d1), _f32),
            jax.ShapeDtypeStruct((e_part, d2), _f32),
        ),
        mesh=_MESH,
        scratch_types=[
            pltpu.VMEM((per_w,), jnp.int32),
            pltpu.VMEM((per_w,), jnp.int32),
            pltpu.VMEM((2, _CH, d1), _f32),
            pltpu.VMEM((2, _CH, d2), _f32),
        ] + [pltpu.SemaphoreType.DMA] * 4,
    )


_EA = 163840
_EB = _E - _EA

_gather_a = _make_gather2(2 * _H, _H, _EA)
_gather_b = _make_gather2(2 * _H, _H, _EB)
_gather_p1_p2 = _make_gather2(_H, _H)


_NROWS = 632
_N_PAD = _NROWS * _NS


def _make_scatter2(e_part):
    def body(np_hbm, sig_hbm, dst_hbm, zero_hbm, num_hbm, den_hbm,
             idx_v, pay_v, acc_sh, si0, si1, sp0, sp1, ss0, ss1):
        cid = lax.axis_index("c")
        sid = lax.axis_index("s")
        nrows = _NROWS
        rows0 = sid * nrows
        per_tile = e_part // _NS
        ebase = sid * per_tile

        pltpu.sync_copy(zero_hbm.at[pl.ds(rows0, nrows)],
                        acc_sh.at[pl.ds(rows0, nrows)])
        plsc.subcore_barrier()

        n_chunks = per_tile // _CH

        def scatter_from(src_hbm):
            si = (si0, si1)
            sp = (sp0, sp1)
            ss = (ss0, ss1)

            def load(i, b):
                off = pl.multiple_of(ebase + i * _CH, 8)
                pltpu.async_copy(dst_hbm.at[pl.ds(off, _CH)], idx_v.at[b], si[b])
                pltpu.async_copy(src_hbm.at[pl.ds(off, _CH)], pay_v.at[b], sp[b])

            def load_wait(i, b):
                off = pl.multiple_of(ebase + i * _CH, 8)
                pltpu.make_async_copy(
                    dst_hbm.at[pl.ds(off, _CH)], idx_v.at[b], si[b]).wait()
                pltpu.make_async_copy(
                    src_hbm.at[pl.ds(off, _CH)], pay_v.at[b], sp[b]).wait()

            def scat_issue(b):
                pltpu.async_copy(pay_v.at[b], acc_sh.at[idx_v.at[b]], ss[b],
                                 add=True)

            def scat_wait(b):
                pltpu.make_async_copy(pay_v.at[b], acc_sh.at[idx_v.at[b]],
                                      ss[b]).wait()

            def maybe(cond, fn):
                if cond is True:
                    fn()
                elif cond is not False:
                    pl.when(cond)(fn)

            def stage(i, b, has_prev, has_next):
                load_wait(i, b)
                scat_issue(b)
                maybe(has_prev, lambda: scat_wait(1 - b))
                maybe(has_next, lambda: load(i + 1, 1 - b))

            load(0, 0)

            @pl.loop(0, n_chunks // 2)
            def _(j):
                i0 = j * 2
                stage(i0, 0, j > 0, True)
                stage(i0 + 1, 1, True, i0 + 2 < n_chunks)

            scat_wait(1)

        @pl.when(cid == 0)
        def _():
            scatter_from(np_hbm)

        @pl.when(cid == 1)
        def _():
            scatter_from(sig_hbm)

        plsc.subcore_barrier()

        @pl.when(cid == 0)
        def _():
            pltpu.sync_copy(acc_sh.at[pl.ds(rows0, nrows)],
                            num_hbm.at[pl.ds(rows0, nrows)])

        @pl.when(cid == 1)
        def _():
            pltpu.sync_copy(acc_sh.at[pl.ds(rows0, nrows)],
                            den_hbm.at[pl.ds(rows0, nrows)])


    return pl.kernel(
        body,
        out_type=(
            jax.ShapeDtypeStruct((_N_PAD, _H), _f32),
            jax.ShapeDtypeStruct((_N_PAD, _H), _f32),
        ),
        mesh=_MESH,
        scratch_types=[
            pltpu.VMEM((2, _CH), jnp.int32),
            pltpu.VMEM((2, _CH, _H), _f32),
            pltpu.VMEM_SHARED((_N_PAD, _H), _f32),
        ] + [pltpu.SemaphoreType.DMA] * 6,
    )


_scatter_a = _make_scatter2(_EA)
_scatter_b = _make_scatter2(_EB)


def kernel(h, e, edge_index, params):
    src = edge_index[0]
    dst = edge_index[1]
    r2 = lambda b: b.reshape(1, -1)

    h = _mm(h, params['emb_h'][0], r2(params['emb_h'][1]), blk=2000)
    zeros = jnp.zeros((_N_PAD, _H), _f32)

    we, be = params['emb_e']
    z = e.reshape(_E // 8, _H)
    eye8 = jnp.eye(8, dtype=_f32)
    src_a, src_b = src[:_EA], src[_EA:]
    dst_a, dst_b = dst[:_EA], dst[_EA:]
    zoff_b = _EA // 1280

    qa = qb = None
    for lp in params['layers']:
        wc, bc = lp['C']
        wz = jnp.kron(eye8, we @ wc)
        bp = be @ wc + bc
        ah, db, eh = _node_tf(h, lp)
        gdba, gea = _gather_a(db, eh, src_a, dst_a)
        gdbb, geb = _gather_b(db, eh, src_b, dst_b)
        if qa is None:
            qa, siga, npa = _edge_fuse1(z, gdba, gea, wz, bp, _EA, 0)
            qb, sigb, npb = _edge_fuse1(z, gdbb, geb, wz, bp, _EB, zoff_b)
        else:
            siga, npa = _edge_fuse2(z, qa, gdba, gea, wz, wc, bp, _EA, 0)
            sigb, npb = _edge_fuse2(z, qb, gdbb, geb, wz, wc, bp, _EB,
                                    zoff_b)
        numa, dena = _scatter_a(npa, siga, dst_a, zeros)
        numb, denb = _scatter_b(npb, sigb, dst_b, zeros)
        h = _h_update(h, ah, numa, dena, numb, denb)

    w1, b1 = params['mlp_e'][0]
    hn, p1, p2 = _node_ro(h, params['mlp_n'], w1[:_H], w1[_H:], b1)
    g1, g2 = _gather_p1_p2(p1, p2, src, dst)
    ef = _edge_mlp(g1, g2, params['mlp_e'][1], params['mlp_e'][2])
    return hn, ef


# speedup vs baseline: 1.3165x; 1.0361x over previous
import functools

import jax
import jax.numpy as jnp
from jax import lax
from jax.experimental import pallas as pl
from jax.experimental.pallas import tpu as pltpu
from jax.experimental.pallas import tpu_sc as plsc

_N = 10000
_E = 320000
_H = 128
_NC = 2
_NS = 16
_NW = _NC * _NS
_CH = 80

_f32 = jnp.float32


def _dot(a, b):
    return jnp.dot(a, b, preferred_element_type=_f32)


def _mm_bias_kernel(x_ref, w_ref, b_ref, o_ref):
    o_ref[...] = _dot(x_ref[...], w_ref[...]) + b_ref[...]


def _mm(x, w, b, blk):
    m, k = x.shape
    n = w.shape[1]
    return pl.pallas_call(
        _mm_bias_kernel,
        grid=(m // blk,),
        in_specs=[
            pl.BlockSpec((blk, k), lambda i: (i, 0)),
            pl.BlockSpec((k, n), lambda i: (0, 0)),
            pl.BlockSpec((1, n), lambda i: (0, 0)),
        ],
        out_specs=pl.BlockSpec((blk, n), lambda i: (i, 0)),
        out_shape=jax.ShapeDtypeStruct((m, n), _f32),
    )(x, w, b)


def _node_tf_kernel(h_ref, wa, ba, wb, bb, wd, bd, we, be,
                    ah_ref, db_ref, eh_ref):
    h = h_ref[...]
    ah_ref[...] = _dot(h, wa[...]) + ba[...]
    db_ref[:, :_H] = _dot(h, wd[...]) + bd[...]
    db_ref[:, _H:] = _dot(h, wb[...]) + bb[...]
    eh_ref[...] = _dot(h, we[...]) + be[...]


def _node_tf(h, lp, blk=2000):
    wspec = pl.BlockSpec((_H, _H), lambda i: (0, 0))
    bspec = pl.BlockSpec((1, _H), lambda i: (0, 0))
    r2 = lambda b: b.reshape(1, _H)
    return pl.pallas_call(
        _node_tf_kernel,
        grid=(_N // blk,),
        in_specs=[pl.BlockSpec((blk, _H), lambda i: (i, 0))]
        + [wspec, bspec] * 4,
        out_specs=[
            pl.BlockSpec((blk, _H), lambda i: (i, 0)),
            pl.BlockSpec((blk, 2 * _H), lambda i: (i, 0)),
            pl.BlockSpec((blk, _H), lambda i: (i, 0)),
        ],
        out_shape=[
            jax.ShapeDtypeStruct((_N, _H), _f32),
            jax.ShapeDtypeStruct((_N, 2 * _H), _f32),
            jax.ShapeDtypeStruct((_N, _H), _f32),
        ],
    )(h, lp['A'][0], r2(lp['A'][1]), lp['B'][0], r2(lp['B'][1]),
      lp['D'][0], r2(lp['D'][1]), lp['E'][0], r2(lp['E'][1]))


def _edge_fuse1_kernel(z_ref, gdb_ref, ge_ref, wz, bp,
                       q_ref, sig_ref, np_ref):
    blk = gdb_ref.shape[0]
    ce = _dot(z_ref[...], wz[...]).reshape(blk, _H) + bp[...]
    en = ce + gdb_ref[:, :_H] + ge_ref[...]
    sig = jax.nn.sigmoid(en)
    q_ref[...] = jnp.maximum(en, 0.0)
    sig_ref[...] = sig
    np_ref[...] = sig * gdb_ref[:, _H:]


def _edge_fuse1(z, gdb, ge, wz, bp, e_part, zoff, blk=1280):
    espec = pl.BlockSpec((blk, _H), lambda i: (i, 0))
    return pl.pallas_call(
        _edge_fuse1_kernel,
        grid=(e_part // blk,),
        in_specs=[
            pl.BlockSpec((blk // 8, _H), lambda i, z0=zoff: (i + z0, 0)),
            pl.BlockSpec((blk, 2 * _H), lambda i: (i, 0)),
            espec,
            pl.BlockSpec((_H, 8 * _H), lambda i: (0, 0)),
            pl.BlockSpec((1, _H), lambda i: (0, 0)),
        ],
        out_specs=[espec, espec, espec],
        out_shape=[jax.ShapeDtypeStruct((e_part, _H), _f32)] * 3,
    )(z, gdb, ge, wz, bp.reshape(1, _H))


def _edge_fuse2_kernel(z_ref, q_ref, gdb_ref, ge_ref, wz, wc, bp,
                       sig_ref, np_ref):
    blk = gdb_ref.shape[0]
    ce = _dot(z_ref[...], wz[...]).reshape(blk, _H) + bp[...]
    ce = ce + _dot(q_ref[...], wc[...])
    en = ce + gdb_ref[:, :_H] + ge_ref[...]
    sig = jax.nn.sigmoid(en)
    sig_ref[...] = sig
    np_ref[...] = sig * gdb_ref[:, _H:]


def _edge_fuse2(z, q, gdb, ge, wz, wc, bp, e_part, zoff, blk=1280):
    espec = pl.BlockSpec((blk, _H), lambda i: (i, 0))
    return pl.pallas_call(
        _edge_fuse2_kernel,
        grid=(e_part // blk,),
        in_specs=[
            pl.BlockSpec((blk // 8, _H), lambda i, z0=zoff: (i + z0, 0)),
            espec,
            pl.BlockSpec((blk, 2 * _H), lambda i: (i, 0)),
            espec,
            pl.BlockSpec((_H, 8 * _H), lambda i: (0, 0)),
            pl.BlockSpec((_H, _H), lambda i: (0, 0)),
            pl.BlockSpec((1, _H), lambda i: (0, 0)),
        ],
        out_specs=[espec, espec],
        out_shape=[jax.ShapeDtypeStruct((e_part, _H), _f32)] * 2,
    )(z, q, gdb, ge, wz, wc, bp.reshape(1, _H))


def _h_update_kernel(h_ref, ah_ref, n0, d0, n1, d1, n2, d2, o_ref):
    num = n0[...] + n1[...] + n2[...]
    den = d0[...] + d1[...] + d2[...]
    o_ref[...] = h_ref[...] + jnp.maximum(
        ah_ref[...] + num / (den + 1e-6), 0.0)


def _h_update(h, ah, partials, blk=2000):
    spec = pl.BlockSpec((blk, _H), lambda i: (i, 0))
    return pl.pallas_call(
        _h_update_kernel,
        grid=(_N // blk,),
        in_specs=[spec] * 8,
        out_specs=spec,
        out_shape=jax.ShapeDtypeStruct((_N, _H), _f32),
    )(h, ah, *partials)


def _node_ro_kernel(h_ref, w1, b1, w2, b2, w3, b3, wea, web, beb,
                    hn_ref, p1_ref, p2_ref):
    h = h_ref[...]
    t = jnp.maximum(_dot(h, w1[...]) + b1[...], 0.0)
    t = jnp.maximum(_dot(t, w2[...]) + b2[...], 0.0)
    hn_ref[...] = _dot(t, w3[...]) + b3[...]
    p1_ref[...] = _dot(h, wea[...])
    p2_ref[...] = _dot(h, web[...]) + beb[...]


def _node_ro(h, mlp_n, wea, web, beb, blk=2000):
    specs = []
    args = [h]
    for (w, b) in mlp_n:
        k, n = w.shape
        specs += [pl.BlockSpec((k, n), lambda i: (0, 0)),
                  pl.BlockSpec((1, n), lambda i: (0, 0))]
        args += [w, b.reshape(1, n)]
    specs += [pl.BlockSpec((_H, _H), lambda i: (0, 0))] * 2
    specs += [pl.BlockSpec((1, _H), lambda i: (0, 0))]
    args += [wea, web, beb.reshape(1, _H)]
    hspec = pl.BlockSpec((blk, _H), lambda i: (i, 0))
    return pl.pallas_call(
        _node_ro_kernel,
        grid=(_N // blk,),
        in_specs=[hspec] + specs,
        out_specs=[hspec, hspec, hspec],
        out_shape=[jax.ShapeDtypeStruct((_N, _H), _f32)] * 3,
    )(*args)


def _edge_mlp_kernel(g1_ref, g2_ref, w2, b2, w3, b3, o_ref):
    g = jnp.maximum(g1_ref[...] + g2_ref[...], 0.0)
    t = jnp.maximum(_dot(g, w2[...]) + b2[...], 0.0)
    o_ref[...] = _dot(t, w3[...]) + b3[...]


def _edge_mlp(g1, g2, l2, l3, blk=1280):
    w2, b2 = l2
    w3, b3 = l3
    espec = pl.BlockSpec((blk, _H), lambda i: (i, 0))
    return pl.pallas_call(
        _edge_mlp_kernel,
        grid=(_E // blk,),
        in_specs=[
            espec, espec,
            pl.BlockSpec(w2.shape, lambda i: (0, 0)),
            pl.BlockSpec((1, w2.shape[1]), lambda i: (0, 0)),
            pl.BlockSpec(w3.shape, lambda i: (0, 0)),
            pl.BlockSpec((1, w3.shape[1]), lambda i: (0, 0)),
        ],
        out_specs=espec,
        out_shape=jax.ShapeDtypeStruct((_E, _H), _f32),
    )(g1, g2, w2, b2.reshape(1, -1), w3, b3.reshape(1, -1))


_MESH = plsc.VectorSubcoreMesh(core_axis_name="c", subcore_axis_name="s")


def _make_gather2(d1, d2, e_part=_E):
    per_w = e_part // _NW
    n_chunks = per_w // _CH

    def body(t1, t2, i1_hbm, i2_hbm, o1, o2,
             i1_v, i2_v, r1_v, r2_v, sg0, sg1, so0, so1):
        wid = lax.axis_index("s") * _NC + lax.axis_index("c")
        base = wid * per_w
        sg = (sg0, sg1)
        so = (so0, so1)

        pltpu.sync_copy(i1_hbm.at[pl.ds(base, per_w)], i1_v)
        pltpu.sync_copy(i2_hbm.at[pl.ds(base, per_w)], i2_v)

        def gather_issue(i, b):
            loff = pl.multiple_of(i * _CH, 8)
            pltpu.async_copy(t1.at[i1_v.at[pl.ds(loff, _CH)]],
                             r1_v.at[b], sg[b])
            pltpu.async_copy(t2.at[i2_v.at[pl.ds(loff, _CH)]],
                             r2_v.at[b], sg[b])

        def gather_wait(i, b):
            loff = pl.multiple_of(i * _CH, 8)
            pltpu.make_async_copy(t1.at[i1_v.at[pl.ds(loff, _CH)]],
                                  r1_v.at[b], sg[b]).wait()
            pltpu.make_async_copy(t2.at[i2_v.at[pl.ds(loff, _CH)]],
                                  r2_v.at[b], sg[b]).wait()

        def write_issue(i, b):
            off = pl.multiple_of(base + i * _CH, 8)
            pltpu.async_copy(r1_v.at[b], o1.at[pl.ds(off, _CH)], so[b])
            pltpu.async_copy(r2_v.at[b], o2.at[pl.ds(off, _CH)], so[b])

        def write_wait(i, b):
            off = pl.multiple_of(base + i * _CH, 8)
            pltpu.make_async_copy(
                r1_v.at[b], o1.at[pl.ds(off, _CH)], so[b]).wait()
            pltpu.make_async_copy(
                r2_v.at[b], o2.at[pl.ds(off, _CH)], so[b]).wait()

        def maybe(cond, fn):
            if cond is True:
                fn()
            elif cond is not False:
                pl.when(cond)(fn)

        def stage(i, b, has_prev, has_prev2):
            maybe(has_prev2, lambda: write_wait(i - 2, b))
            gather_issue(i, b)

            def drain_prev():
                gather_wait(i - 1, 1 - b)
                write_issue(i - 1, 1 - b)
            maybe(has_prev, drain_prev)

        @pl.loop(0, n_chunks // 2)
        def _(j):
            i0 = j * 2
            stage(i0, 0, j > 0, j > 0)
            stage(i0 + 1, 1, True, j > 0)

        last = n_chunks - 1
        if n_chunks % 2 == 1:
            write_wait(last - 2, 0)
            gather_issue(last, 0)
            gather_wait(last - 1, 1)
            write_issue(last - 1, 1)
            gather_wait(last, 0)
            write_issue(last, 0)
            write_wait(last - 1, 1)
            write_wait(last, 0)
        else:
            gather_wait(last, 1)
            write_issue(last, 1)
            write_wait(last - 1, 0)
            write_wait(last, 1)

    return pl.kernel(
        body,
        out_type=(
            jax.ShapeDtypeStruct((e_part, d1), _f32),
            jax.ShapeDtypeStruct((e_part, d2), _f32),
        ),
        mesh=_MESH,
        scratch_types=[
            pltpu.VMEM((per_w,), jnp.int32),
            pltpu.VMEM((per_w,), jnp.int32),
            pltpu.VMEM((2, _CH, d1), _f32),
            pltpu.VMEM((2, _CH, d2), _f32),
        ] + [pltpu.SemaphoreType.DMA] * 4,
    )


_PARTS = (104960, 104960, 110080)

_gather_parts = [_make_gather2(2 * _H, _H, ep) for ep in _PARTS]
_gather_p1_p2 = _make_gather2(_H, _H)


_NROWS = 632
_N_PAD = _NROWS * _NS


def _make_scatter2(e_part):
    def body(np_hbm, sig_hbm, dst_hbm, zero_hbm, num_hbm, den_hbm,
             idx_v, pay_v, acc_sh, si0, si1, sp0, sp1, ss0, ss1):
        cid = lax.axis_index("c")
        sid = lax.axis_index("s")
        nrows = _NROWS
        rows0 = sid * nrows
        per_tile = e_part // _NS
        ebase = sid * per_tile

        pltpu.sync_copy(zero_hbm.at[pl.ds(rows0, nrows)],
                        acc_sh.at[pl.ds(rows0, nrows)])
        plsc.subcore_barrier()

        n_chunks = per_tile // _CH

        def scatter_from(src_hbm):
            si = (si0, si1)
            sp = (sp0, sp1)
            ss = (ss0, ss1)

            def load(i, b):
                off = pl.multiple_of(ebase + i * _CH, 8)
                pltpu.async_copy(dst_hbm.at[pl.ds(off, _CH)], idx_v.at[b], si[b])
                pltpu.async_copy(src_hbm.at[pl.ds(off, _CH)], pay_v.at[b], sp[b])

            def load_wait(i, b):
                off = pl.multiple_of(ebase + i * _CH, 8)
                pltpu.make_async_copy(
                    dst_hbm.at[pl.ds(off, _CH)], idx_v.at[b], si[b]).wait()
                pltpu.make_async_copy(
                    src_hbm.at[pl.ds(off, _CH)], pay_v.at[b], sp[b]).wait()

            def scat_issue(b):
                pltpu.async_copy(pay_v.at[b], acc_sh.at[idx_v.at[b]], ss[b],
                                 add=True)

            def scat_wait(b):
                pltpu.make_async_copy(pay_v.at[b], acc_sh.at[idx_v.at[b]],
                                      ss[b]).wait()

            def maybe(cond, fn):
                if cond is True:
                    fn()
                elif cond is not False:
                    pl.when(cond)(fn)

            def stage(i, b, has_prev, has_next):
                load_wait(i, b)
                scat_issue(b)
                maybe(has_prev, lambda: scat_wait(1 - b))
                maybe(has_next, lambda: load(i + 1, 1 - b))

            load(0, 0)

            @pl.loop(0, n_chunks // 2)
            def _(j):
                i0 = j * 2
                stage(i0, 0, j > 0, True)
                stage(i0 + 1, 1, True, i0 + 2 < n_chunks)

            scat_wait(1)

        @pl.when(cid == 0)
        def _():
            scatter_from(np_hbm)

        @pl.when(cid == 1)
        def _():
            scatter_from(sig_hbm)

        plsc.subcore_barrier()

        @pl.when(cid == 0)
        def _():
            pltpu.sync_copy(acc_sh.at[pl.ds(rows0, nrows)],
                            num_hbm.at[pl.ds(rows0, nrows)])

        @pl.when(cid == 1)
        def _():
            pltpu.sync_copy(acc_sh.at[pl.ds(rows0, nrows)],
                            den_hbm.at[pl.ds(rows0, nrows)])


    return pl.kernel(
        body,
        out_type=(
            jax.ShapeDtypeStruct((_N_PAD, _H), _f32),
            jax.ShapeDtypeStruct((_N_PAD, _H), _f32),
        ),
        mesh=_MESH,
        scratch_types=[
            pltpu.VMEM((2, _CH), jnp.int32),
            pltpu.VMEM((2, _CH, _H), _f32),
            pltpu.VMEM_SHARED((_N_PAD, _H), _f32),
        ] + [pltpu.SemaphoreType.DMA] * 6,
    )


_scatter_parts = [_make_scatter2(ep) for ep in _PARTS]


def kernel(h, e, edge_index, params):
    src = edge_index[0]
    dst = edge_index[1]
    r2 = lambda b: b.reshape(1, -1)

    h = _mm(h, params['emb_h'][0], r2(params['emb_h'][1]), blk=2000)
    zeros = jnp.zeros((_N_PAD, _H), _f32)

    we, be = params['emb_e']
    z = e.reshape(_E // 8, _H)
    eye8 = jnp.eye(8, dtype=_f32)
    offs = [0]
    for ep in _PARTS:
        offs.append(offs[-1] + ep)
    src_p = [src[offs[k]:offs[k + 1]] for k in range(len(_PARTS))]
    dst_p = [dst[offs[k]:offs[k + 1]] for k in range(len(_PARTS))]
    zoffs = [offs[k] // 1280 for k in range(len(_PARTS))]

    qs = None
    for lp in params['layers']:
        wc, bc = lp['C']
        wz = jnp.kron(eye8, we @ wc)
        bp = be @ wc + bc
        ah, db, eh = _node_tf(h, lp)
        gath = [_gather_parts[k](db, eh, src_p[k], dst_p[k])
                for k in range(len(_PARTS))]
        if qs is None:
            fused = [_edge_fuse1(z, gath[k][0], gath[k][1], wz, bp,
                                 _PARTS[k], zoffs[k])
                     for k in range(len(_PARTS))]
            qs = [f[0] for f in fused]
            fused = [(f[1], f[2]) for f in fused]
        else:
            fused = [_edge_fuse2(z, qs[k], gath[k][0], gath[k][1], wz, wc,
                                 bp, _PARTS[k], zoffs[k])
                     for k in range(len(_PARTS))]
        partials = []
        for k in range(len(_PARTS)):
            sig_k, np_k = fused[k]
            num_k, den_k = _scatter_parts[k](np_k, sig_k, dst_p[k], zeros)
            partials += [num_k, den_k]
        h = _h_update(h, ah, partials)

    w1, b1 = params['mlp_e'][0]
    hn, p1, p2 = _node_ro(h, params['mlp_n'], w1[:_H], w1[_H:], b1)
    g1, g2 = _gather_p1_p2(p1, p2, src, dst)
    ef = _edge_mlp(g1, g2, params['mlp_e'][1], params['mlp_e'][2])
    return hn, ef
